# baseline (speedup 1.0000x reference)
import functools

import jax
import jax.numpy as jnp
from jax import lax
from jax.experimental import pallas as pl
from jax.experimental.pallas import tpu as pltpu
from jax.experimental.pallas import tpu_sc as plsc

D = 128
L = 16
FC = D // L
NC = 2
NSUB = 16
NW = NC * NSUB

N_NODES = 10000
N_EDGES = 320000
N_SUPER = 1000
N_BIP = 40000
N_SED = 16000
NBP = 40960
NSP = 16384
CH = 128

@functools.cache
def _mesh():
    return plsc.VectorSubcoreMesh(
        core_axis_name="c", subcore_axis_name="s",
        num_cores=NC, num_subcores=NSUB)


_f32 = jnp.float32
_i32 = jnp.int32


def _zero_vmem(buf, nrows):
    z = jnp.zeros((L,), _f32)

    @pl.loop(0, nrows)
    def _(r):
        for j in range(FC):
            buf[r, pl.ds(j * L, L)] = z


def _scale_rows(rows, wv, nrows):
    @pl.loop(0, nrows)
    def _(r):
        wr = wv[r, :]
        for j in range(FC):
            rows[r, pl.ds(j * L, L)] = rows[r, pl.ds(j * L, L)] * wr


@functools.cache
def _build_sc1():
    return functools.partial(
        pl.kernel,
        out_type=(jax.ShapeDtypeStruct((NC, 1024, D), _f32),
                  jax.ShapeDtypeStruct((NC, 1024, D), _f32)),
        mesh=_mesh(),
        scratch_types=[
            [pltpu.VMEM((CH,), _i32)] * 2,
            [pltpu.VMEM((CH,), _i32)] * 2,
            [pltpu.VMEM((CH, L), _f32)] * 2,
            [pltpu.VMEM((CH, D), _f32)] * 2,
            pltpu.VMEM((64, D), _f32),
            pltpu.VMEM_SHARED((1024, D), _f32),
            pltpu.VMEM_SHARED((1024, D), _f32),
            pltpu.SemaphoreType.DMA,
            [pltpu.SemaphoreType.DMA] * 2,
            [pltpu.SemaphoreType.DMA] * 2,
            [pltpu.SemaphoreType.DMA] * 2,
            [pltpu.SemaphoreType.DMA] * 2,
        ],
    )(_sc1_body)


def _sc1_body(nodes_hbm, b0_hbm, b1_hbm, bw_hbm, sed_hbm, sw_hbm, sg1_hbm,
              nmsg_out, amsg_out, sidx, didx, wv, rows, zbuf, acc_n, acc_a,
              sem, si, di, ws, la):
    c = lax.axis_index("c")
    s = lax.axis_index("s")
    wid = s * NC + c

    _zero_vmem(zbuf, 64)
    pltpu.sync_copy(zbuf, acc_n.at[pl.ds(s * 64, 64)])
    pltpu.sync_copy(zbuf, acc_a.at[pl.ds(s * 64, 64)])
    plsc.subcore_barrier()

    nb_w = NBP // NW
    base_b = wid * nb_w
    n_a = nb_w // CH

    def pref_a(i, b):
        base = base_b + i * CH
        pltpu.async_copy(b0_hbm.at[pl.ds(base, CH)], sidx[b], si[b])
        pltpu.async_copy(b1_hbm.at[pl.ds(base, CH)], didx[b], di[b])
        pltpu.async_copy(bw_hbm.at[pl.ds(base, CH)], wv[b], ws[b])

    def drain_a(b):
        pltpu.make_async_copy(b0_hbm.at[pl.ds(0, CH)], sidx[b], si[b]).wait()
        pltpu.make_async_copy(b1_hbm.at[pl.ds(0, CH)], didx[b], di[b]).wait()
        pltpu.make_async_copy(bw_hbm.at[pl.ds(0, CH)], wv[b], ws[b]).wait()

    pref_a(0, 0)

    @pl.loop(0, n_a, step=2)
    def _(g):
        for b in range(2):
            i = g + b
            nb = 1 - b
            if b == 0:
                pref_a(i + 1, nb)
            else:
                @pl.when(g < n_a - 2)
                def _():
                    pref_a(i + 1, nb)
            drain_a(b)
            pltpu.async_copy(nodes_hbm.at[sidx[b]], rows[b], sem).wait()
            _scale_rows(rows[b], wv[b], CH)
            pltpu.sync_copy(rows[b], acc_n.at[didx[b]], add=True)

    ns_w = NSP // NW
    base_s = wid * ns_w
    n_b = ns_w // CH

    def pref_b(i, b):
        base = base_s + i * CH
        pltpu.async_copy(sed_hbm.at[pl.ds(base, CH)], rows[b], la[b])
        pltpu.async_copy(sg1_hbm.at[pl.ds(base, CH)], didx[b], di[b])
        pltpu.async_copy(sw_hbm.at[pl.ds(base, CH)], wv[b], ws[b])

    def drain_b(b):
        pltpu.make_async_copy(sed_hbm.at[pl.ds(0, CH)], rows[b], la[b]).wait()
        pltpu.make_async_copy(sg1_hbm.at[pl.ds(0, CH)], didx[b], di[b]).wait()
        pltpu.make_async_copy(sw_hbm.at[pl.ds(0, CH)], wv[b], ws[b]).wait()

    pref_b(0, 0)

    @pl.loop(0, n_b, step=2)
    def _(g):
        for b in range(2):
            i = g + b
            nb = 1 - b
            if b == 0:
                pref_b(i + 1, nb)
            else:
                @pl.when(g < n_b - 2)
                def _():
                    pref_b(i + 1, nb)
            drain_b(b)
            _scale_rows(rows[b], wv[b], CH)
            pltpu.sync_copy(rows[b], acc_a.at[didx[b]], add=True)

    plsc.subcore_barrier()
    pltpu.sync_copy(acc_n.at[pl.ds(s * 64, 64)], nmsg_out.at[c, pl.ds(s * 64, 64)])
    pltpu.sync_copy(acc_a.at[pl.ds(s * 64, 64)], amsg_out.at[c, pl.ds(s * 64, 64)])


N_ACC = 10112
CH2 = 64
_ED_W = N_EDGES // NW
_ED_FULL = _ED_W // CH
_ED_TAIL = _ED_W - _ED_FULL * CH
_ED2_FULL = _ED_W // CH2


@functools.cache
def _build_sc2():
    return functools.partial(
        pl.kernel,
        out_type=(jax.ShapeDtypeStruct((NC, N_ACC, D), _f32),
                  jax.ShapeDtypeStruct((NC, N_ACC, D), _f32)),
        mesh=_mesh(),
        scratch_types=[
            [pltpu.VMEM((CH2,), _i32)] * 2,
            [pltpu.VMEM((CH2,), _i32)] * 2,
            pltpu.VMEM((_ED_TAIL,), _i32),
            [pltpu.VMEM((CH2, L), _f32)] * 2,
            [pltpu.VMEM((CH2, D), _f32)] * 2,
            pltpu.VMEM_SHARED((N_ACC, D), _f32),
            pltpu.SemaphoreType.DMA,
            [pltpu.SemaphoreType.DMA] * 2,
            [pltpu.SemaphoreType.DMA] * 2,
            [pltpu.SemaphoreType.DMA] * 2,
            [pltpu.SemaphoreType.DMA] * 2,
        ],
    )(_sc2_body)


def _sc2_body(sn_hbm, b0_hbm, b1_hbm, bw_hbm, edges_hbm, g1_hbm,
              smsg_out, emsg_out, sidx, didx, didx_t, wv, rows, acc,
              sem, si, di, ws, la):
    c = lax.axis_index("c")
    s = lax.axis_index("s")
    wid = s * NC + c
    row0 = s * 632

    def zero_acc():
        _zero_vmem(rows[0], CH2)
        for t in range(9):
            pltpu.sync_copy(rows[0], acc.at[pl.ds(row0 + t * 64, 64)])
        pltpu.sync_copy(rows[0].at[pl.ds(0, 56)],
                        acc.at[pl.ds(row0 + 576, 56)])

    zero_acc()
    plsc.subcore_barrier()

    nb_w = NBP // NW
    base_b = wid * nb_w
    n_a = nb_w // CH2

    def pref_a(i, b):
        base = base_b + i * CH2
        pltpu.async_copy(b1_hbm.at[pl.ds(base, CH2)], sidx[b], si[b])
        pltpu.async_copy(b0_hbm.at[pl.ds(base, CH2)], didx[b], di[b])
        pltpu.async_copy(bw_hbm.at[pl.ds(base, CH2)], wv[b], ws[b])

    def drain_a(b):
        pltpu.make_async_copy(b1_hbm.at[pl.ds(0, CH2)], sidx[b], si[b]).wait()
        pltpu.make_async_copy(b0_hbm.at[pl.ds(0, CH2)], didx[b], di[b]).wait()
        pltpu.make_async_copy(bw_hbm.at[pl.ds(0, CH2)], wv[b], ws[b]).wait()

    pref_a(0, 0)

    @pl.loop(0, n_a, step=2)
    def _(g):
        for b in range(2):
            i = g + b
            nb = 1 - b
            if b == 0:
                pref_a(i + 1, nb)
            else:
                @pl.when(g < n_a - 2)
                def _():
                    pref_a(i + 1, nb)
            drain_a(b)
            pltpu.async_copy(sn_hbm.at[sidx[b]], rows[b], sem).wait()
            _scale_rows(rows[b], wv[b], CH2)
            pltpu.sync_copy(rows[b], acc.at[didx[b]], add=True)

    plsc.subcore_barrier()
    pltpu.sync_copy(acc.at[pl.ds(row0, 632)], smsg_out.at[c, pl.ds(row0, 632)])

    zero_acc()
    plsc.subcore_barrier()

    base_e = wid * _ED_W

    def pref_e(i, b):
        base = base_e + i * CH2
        pltpu.async_copy(edges_hbm.at[pl.ds(base, CH2)], rows[b], la[b])
        pltpu.async_copy(g1_hbm.at[pl.ds(base, CH2)], didx[b], di[b])

    def drain_e(b):
        pltpu.make_async_copy(edges_hbm.at[pl.ds(0, CH2)], rows[b],
                              la[b]).wait()
        pltpu.make_async_copy(g1_hbm.at[pl.ds(0, CH2)], didx[b],
                              di[b]).wait()

    pref_e(0, 0)

    @pl.loop(0, _ED2_FULL, step=2)
    def _(g):
        for b in range(2):
            i = g + b
            nb = 1 - b
            if b == 0:
                pref_e(i + 1, nb)
            else:
                @pl.when(g < _ED2_FULL - 2)
                def _():
                    pref_e(i + 1, nb)
            drain_e(b)
            pltpu.sync_copy(rows[b], acc.at[didx[b]], add=True)

    tbase = base_e + _ED2_FULL * CH2
    pltpu.sync_copy(edges_hbm.at[pl.ds(tbase, _ED_TAIL)],
                    rows[0].at[pl.ds(0, _ED_TAIL)])
    pltpu.sync_copy(g1_hbm.at[pl.ds(tbase, _ED_TAIL)], didx_t)
    pltpu.sync_copy(rows[0].at[pl.ds(0, _ED_TAIL)], acc.at[didx_t], add=True)

    plsc.subcore_barrier()
    pltpu.sync_copy(acc.at[pl.ds(row0, 632)], emsg_out.at[c, pl.ds(row0, 632)])


@functools.cache
def _build_sc34():
    return functools.partial(
        pl.kernel,
        out_type=(jax.ShapeDtypeStruct((NSP, D), _f32),
                  jax.ShapeDtypeStruct((N_EDGES, D), _f32),
                  jax.ShapeDtypeStruct((N_EDGES, D), _f32)),
        mesh=_mesh(),
        scratch_types=[
            pltpu.VMEM((_ED_W,), _i32),
            pltpu.VMEM((_ED_W,), _i32),
            [pltpu.VMEM((CH, D), _f32)] * 2,
            [pltpu.VMEM((CH, D), _f32)] * 2,
            [pltpu.SemaphoreType.DMA] * 2,
            [pltpu.SemaphoreType.DMA] * 2,
            [pltpu.SemaphoreType.DMA] * 2,
            [pltpu.SemaphoreType.DMA] * 2,
        ],
    )(_sc34_body)


def _sc34_body(p_hbm, q_hbm, sg0_hbm, sg1_hbm, nodes_hbm, g0_hbm, g1_hbm,
               t_out, na_out, nb_out, idx0, idx1, rowsa, rowsb, ga, gb,
               wa, wb):
    c = lax.axis_index("c")
    s = lax.axis_index("s")
    wid = s * NC + c

    ns_w = NSP // NW
    base_s = wid * ns_w
    n_t = ns_w // CH
    pltpu.sync_copy(sg0_hbm.at[pl.ds(base_s, ns_w)], idx0.at[pl.ds(0, ns_w)])
    pltpu.sync_copy(sg1_hbm.at[pl.ds(base_s, ns_w)], idx1.at[pl.ds(0, ns_w)])

    @pl.loop(0, n_t, step=2)
    def _(g):
        for b in range(2):
            i = g + b
            base = base_s + i * CH
            @pl.when(g > 0)
            def _():
                pltpu.make_async_copy(rowsa[b], t_out.at[pl.ds(0, CH)],
                                      wa[b]).wait()
            cpa = pltpu.async_copy(
                p_hbm.at[idx0.at[pl.ds(i * CH, CH)]], rowsa[b], ga[b])
            cpb = pltpu.async_copy(
                q_hbm.at[idx1.at[pl.ds(i * CH, CH)]], rowsb[b], gb[b])
            cpa.wait()
            cpb.wait()
            ra = rowsa[b]
            rb = rowsb[b]

            @pl.loop(0, CH)
            def _(r):
                for j in range(FC):
                    ra[r, pl.ds(j * L, L)] = (
                        ra[r, pl.ds(j * L, L)] + rb[r, pl.ds(j * L, L)])

            pltpu.async_copy(rowsa[b], t_out.at[pl.ds(base, CH)], wa[b])

    pltpu.make_async_copy(rowsa[0], t_out.at[pl.ds(0, CH)], wa[0]).wait()
    pltpu.make_async_copy(rowsa[1], t_out.at[pl.ds(0, CH)], wa[1]).wait()

    base_e = wid * _ED_W
    pltpu.sync_copy(g0_hbm.at[pl.ds(base_e, _ED_W)], idx0)
    pltpu.sync_copy(g1_hbm.at[pl.ds(base_e, _ED_W)], idx1)

    @pl.loop(0, _ED_FULL, step=2)
    def _(g):
        for b in range(2):
            i = g + b
            nb = 1 - b
            base = base_e + i * CH
            cpa = pltpu.async_copy(
                nodes_hbm.at[idx0.at[pl.ds(i * CH, CH)]], rowsa[b], ga[b])
            cpb = pltpu.async_copy(
                nodes_hbm.at[idx1.at[pl.ds(i * CH, CH)]], rowsb[b], gb[b])

            def drain_prev():
                pltpu.make_async_copy(rowsa[nb], na_out.at[pl.ds(0, CH)],
                                      wa[nb]).wait()
                pltpu.make_async_copy(rowsb[nb], nb_out.at[pl.ds(0, CH)],
                                      wb[nb]).wait()

            if b == 0:
                @pl.when(g > 0)
                def _():
                    drain_prev()
            else:
                drain_prev()
            cpa.wait()
            cpb.wait()
            pltpu.async_copy(rowsa[b], na_out.at[pl.ds(base, CH)], wa[b])
            pltpu.async_copy(rowsb[b], nb_out.at[pl.ds(base, CH)], wb[b])

    pltpu.make_async_copy(rowsa[1], na_out.at[pl.ds(0, CH)], wa[1]).wait()
    pltpu.make_async_copy(rowsb[1], nb_out.at[pl.ds(0, CH)], wb[1]).wait()

    tbase = base_e + _ED_FULL * CH
    pltpu.async_copy(nodes_hbm.at[idx0.at[pl.ds(_ED_FULL * CH, _ED_TAIL)]],
                     rowsa[0].at[pl.ds(0, _ED_TAIL)], ga[0]).wait()
    pltpu.async_copy(nodes_hbm.at[idx1.at[pl.ds(_ED_FULL * CH, _ED_TAIL)]],
                     rowsb[0].at[pl.ds(0, _ED_TAIL)], gb[0]).wait()
    pltpu.sync_copy(rowsa[0].at[pl.ds(0, _ED_TAIL)],
                    na_out.at[pl.ds(tbase, _ED_TAIL)])
    pltpu.sync_copy(rowsb[0].at[pl.ds(0, _ED_TAIL)],
                    nb_out.at[pl.ds(tbase, _ED_TAIL)])


def _dot(a, b):
    return jnp.dot(a, b, preferred_element_type=_f32)


def _sn_body(x, a0, a1, n0, n1, w1x, w1a, w1n, b1, w2, b2, pa, pb, bp,
             xo, po, qo):
    att = a0[0] + a1[0]
    nm = n0[0] + n1[0]
    h = jnp.maximum(
        _dot(x[...], w1x[...]) + _dot(att, w1a[...]) + _dot(nm, w1n[...])
        + b1[...], 0.0)
    xn = jnp.maximum(_dot(h, w2[...]) + b2[...], 0.0) + x[...]
    xo[...] = xn
    po[...] = _dot(xn, pa[...]) + bp[...]
    qo[...] = _dot(xn, pb[...])


def _nn_body(x, e0, e1, s0, s1, w1x, w1e, w1s, b1, w2, b2, xo):
    em = e0[0] + e1[0]
    sm = s0[0] + s1[0]
    h = jnp.maximum(
        _dot(x[...], w1x[...]) + _dot(em, w1e[...]) + _dot(sm, w1s[...])
        + b1[...], 0.0)
    xo[...] = jnp.maximum(_dot(h, w2[...]) + b2[...], 0.0) + x[...]


def _dotb(a, b):
    return jnp.dot(a.astype(jnp.bfloat16), b.astype(jnp.bfloat16),
                   preferred_element_type=_f32)


def _se_body(t, e, c, w2, b2, out):
    h = jnp.maximum(t[...] + _dotb(e[...], c[...]), 0.0)
    out[...] = jnp.tanh(_dotb(h, w2[...]) + b2[...]) + e[...]


def _en_body(na, nb, e, w1a, w1b, w1c, b1, w2, b2, out):
    h = jnp.maximum(
        _dotb(na[...], w1a[...]) + _dotb(nb[...], w1b[...])
        + _dotb(e[...], w1c[...]) + b1[...], 0.0)
    out[...] = jnp.tanh(_dotb(h, w2[...]) + b2[...]) + e[...]


def _row_spec(rows):
    return pl.BlockSpec((rows, D), lambda i: (i, 0))


def _w_spec(shape):
    return pl.BlockSpec(shape, lambda i: tuple(0 for _ in shape))


def _part_spec(rows, core):
    return pl.BlockSpec((1, rows, D), lambda i, _c=core: (_c, i, 0))


def _tc_node_mlp(n, rows, x, msg1, msg2, w1, b1, w2, b2):
    grid = (n // rows,)
    ws = _w_spec((D, D))
    bs = _w_spec((1, D))
    ps0 = _part_spec(rows, 0)
    ps1 = _part_spec(rows, 1)
    return pl.pallas_call(
        _nn_body,
        grid=grid,
        in_specs=[_row_spec(rows), ps0, ps1, ps0, ps1, ws, ws, ws, bs, ws, bs],
        out_specs=_row_spec(rows),
        out_shape=jax.ShapeDtypeStruct((n, D), _f32),
        compiler_params=pltpu.CompilerParams(
            dimension_semantics=("arbitrary",)),
    )(x, msg1, msg1, msg2, msg2, w1[:D], w1[D:2 * D], w1[2 * D:],
      b1.reshape(1, D), w2, b2.reshape(1, D))


def _tc_sn_mlp(x, amsg, nmsg, w1, b1, w2, b2, pa, pb, bp):
    outs = [jax.ShapeDtypeStruct((N_SUPER, D), _f32)] * 3
    xs = pl.BlockSpec((N_SUPER, D), lambda i: (0, 0))
    ws = pl.BlockSpec((D, D), lambda i: (0, 0))
    bs = pl.BlockSpec((1, D), lambda i: (0, 0))
    ps0 = pl.BlockSpec((1, N_SUPER, D), lambda i: (0, 0, 0))
    ps1 = pl.BlockSpec((1, N_SUPER, D), lambda i: (1, 0, 0))
    return pl.pallas_call(
        _sn_body,
        grid=(1,),
        in_specs=[xs, ps0, ps1, ps0, ps1, ws, ws, ws, bs, ws, bs, ws, ws, bs],
        out_specs=[xs] * 3,
        out_shape=outs,
    )(x, amsg, amsg, nmsg, nmsg, w1[:D], w1[D:2 * D], w1[2 * D:],
      b1.reshape(1, D), w2, b2.reshape(1, D), pa, pb, bp.reshape(1, D))


def _tc_se_mlp(t, e, c, w2, b2):
    rows = 2000
    grid = (N_SED // rows,)
    ws = _w_spec((D, D))
    bs = _w_spec((1, D))
    return pl.pallas_call(
        _se_body,
        grid=grid,
        in_specs=[_row_spec(rows), _row_spec(rows), ws, ws, bs],
        out_specs=_row_spec(rows),
        out_shape=jax.ShapeDtypeStruct((N_SED, D), _f32),
        compiler_params=pltpu.CompilerParams(
            dimension_semantics=("arbitrary",)),
    )(t, e, c, w2, b2.reshape(1, D))


def _tc_en_mlp(na, nb, e, w1, b1, w2, b2):
    rows = 2000
    grid = (N_EDGES // rows,)
    ws = _w_spec((D, D))
    bs = _w_spec((1, D))
    return pl.pallas_call(
        _en_body,
        grid=grid,
        in_specs=[_row_spec(rows)] * 3 + [ws, ws, ws, bs, ws, bs],
        out_specs=_row_spec(rows),
        out_shape=jax.ShapeDtypeStruct((N_EDGES, D), _f32),
        compiler_params=pltpu.CompilerParams(
            dimension_semantics=("arbitrary",)),
    )(na, nb, e, w1[:D], w1[D:2 * D], w1[2 * D:], b1.reshape(1, D),
      w2, b2.reshape(1, D))


def kernel(nodes, edges, supernodes, superedges, graph, bipartite_graph,
           bipartite_edge_weights, super_graph, super_edge_weights,
           en_W1, en_b1, en_W2, en_b2, nn_W1, nn_b1, nn_W2, nn_b2,
           sn_W1, sn_b1, sn_W2, sn_b2, se_W1, se_b1, se_W2, se_b2):
    g0 = graph[0]
    g1 = graph[1]
    b0 = jnp.pad(bipartite_graph[0], (0, NBP - N_BIP))
    b1i = jnp.pad(bipartite_graph[1], (0, NBP - N_BIP))
    bw = jnp.broadcast_to(
        jnp.pad(bipartite_edge_weights, ((0, NBP - N_BIP), (0, 0))), (NBP, L))
    sedp = jnp.pad(superedges, ((0, NSP - N_SED), (0, 0)))
    sw = jnp.broadcast_to(
        jnp.pad(super_edge_weights, ((0, NSP - N_SED), (0, 0))), (NSP, L))
    sg0 = jnp.pad(super_graph[0], (0, NSP - N_SED))
    sg1 = jnp.pad(super_graph[1], (0, NSP - N_SED))

    nmsg, amsg = _build_sc1()(nodes, b0, b1i, bw, sedp, sw, sg1)

    sn_new, p_se, q_se = _tc_sn_mlp(
        supernodes, amsg, nmsg, sn_W1, sn_b1, sn_W2, sn_b2,
        se_W1[:D], se_W1[D:2 * D], se_b1)

    smsg, emsg = _build_sc2()(sn_new, b0, b1i, bw, edges, g1)

    nodes_new = _tc_node_mlp(
        N_NODES, 1000, nodes, emsg, smsg, nn_W1, nn_b1, nn_W2, nn_b2)

    t_se, na, nb = _build_sc34()(p_se, q_se, sg0, sg1, nodes_new, g0, g1)
    sed_new = _tc_se_mlp(t_se, superedges, se_W1[2 * D:], se_W2, se_b2)
    edges_new = _tc_en_mlp(na, nb, edges, en_W1, en_b1, en_W2, en_b2)

    return (nodes_new, edges_new, sn_new, sed_new)


# speedup vs baseline: 1.0364x; 1.0364x over previous
import functools

import jax
import jax.numpy as jnp
from jax import lax
from jax.experimental import pallas as pl
from jax.experimental.pallas import tpu as pltpu
from jax.experimental.pallas import tpu_sc as plsc

D = 128
L = 16
FC = D // L
NC = 2
NSUB = 16
NW = NC * NSUB

N_NODES = 10000
N_EDGES = 320000
N_SUPER = 1000
N_BIP = 40000
N_SED = 16000
NBP = 40960
NSP = 16384
CH = 128

@functools.cache
def _mesh():
    return plsc.VectorSubcoreMesh(
        core_axis_name="c", subcore_axis_name="s",
        num_cores=NC, num_subcores=NSUB)


_f32 = jnp.float32
_i32 = jnp.int32


def _zero_vmem(buf, nrows):
    z = jnp.zeros((L,), _f32)

    @pl.loop(0, nrows)
    def _(r):
        for j in range(FC):
            buf[r, pl.ds(j * L, L)] = z


def _scale_rows(rows, wv, nrows):
    @pl.loop(0, nrows)
    def _(r):
        wr = wv[r, :]
        for j in range(FC):
            rows[r, pl.ds(j * L, L)] = rows[r, pl.ds(j * L, L)] * wr


@functools.cache
def _build_sc1():
    return functools.partial(
        pl.kernel,
        out_type=(jax.ShapeDtypeStruct((NC, 1024, D), _f32),
                  jax.ShapeDtypeStruct((NC, 1024, D), _f32)),
        mesh=_mesh(),
        scratch_types=[
            [pltpu.VMEM((CH,), _i32)] * 2,
            [pltpu.VMEM((CH,), _i32)] * 2,
            [pltpu.VMEM((CH, L), _f32)] * 2,
            [pltpu.VMEM((CH, D), _f32)] * 2,
            pltpu.VMEM((64, D), _f32),
            pltpu.VMEM_SHARED((1024, D), _f32),
            pltpu.VMEM_SHARED((1024, D), _f32),
            pltpu.SemaphoreType.DMA,
            [pltpu.SemaphoreType.DMA] * 2,
            [pltpu.SemaphoreType.DMA] * 2,
            [pltpu.SemaphoreType.DMA] * 2,
            [pltpu.SemaphoreType.DMA] * 2,
        ],
    )(_sc1_body)


def _sc1_body(nodes_hbm, b0_hbm, b1_hbm, bw_hbm, sed_hbm, sw_hbm, sg1_hbm,
              nmsg_out, amsg_out, sidx, didx, wv, rows, zbuf, acc_n, acc_a,
              sem, si, di, ws, la):
    c = lax.axis_index("c")
    s = lax.axis_index("s")
    wid = s * NC + c

    _zero_vmem(zbuf, 64)
    pltpu.sync_copy(zbuf, acc_n.at[pl.ds(s * 64, 64)])
    pltpu.sync_copy(zbuf, acc_a.at[pl.ds(s * 64, 64)])
    plsc.subcore_barrier()

    nb_w = NBP // NW
    base_b = wid * nb_w
    n_a = nb_w // CH

    def pref_a(i, b):
        base = base_b + i * CH
        pltpu.async_copy(b0_hbm.at[pl.ds(base, CH)], sidx[b], si[b])
        pltpu.async_copy(b1_hbm.at[pl.ds(base, CH)], didx[b], di[b])
        pltpu.async_copy(bw_hbm.at[pl.ds(base, CH)], wv[b], ws[b])

    def drain_a(b):
        pltpu.make_async_copy(b0_hbm.at[pl.ds(0, CH)], sidx[b], si[b]).wait()
        pltpu.make_async_copy(b1_hbm.at[pl.ds(0, CH)], didx[b], di[b]).wait()
        pltpu.make_async_copy(bw_hbm.at[pl.ds(0, CH)], wv[b], ws[b]).wait()

    pref_a(0, 0)

    @pl.loop(0, n_a, step=2)
    def _(g):
        for b in range(2):
            i = g + b
            nb = 1 - b
            if b == 0:
                pref_a(i + 1, nb)
            else:
                @pl.when(g < n_a - 2)
                def _():
                    pref_a(i + 1, nb)
            drain_a(b)
            pltpu.async_copy(nodes_hbm.at[sidx[b]], rows[b], sem).wait()
            _scale_rows(rows[b], wv[b], CH)
            pltpu.sync_copy(rows[b], acc_n.at[didx[b]], add=True)

    ns_w = NSP // NW
    base_s = wid * ns_w
    n_b = ns_w // CH

    def pref_b(i, b):
        base = base_s + i * CH
        pltpu.async_copy(sed_hbm.at[pl.ds(base, CH)], rows[b], la[b])
        pltpu.async_copy(sg1_hbm.at[pl.ds(base, CH)], didx[b], di[b])
        pltpu.async_copy(sw_hbm.at[pl.ds(base, CH)], wv[b], ws[b])

    def drain_b(b):
        pltpu.make_async_copy(sed_hbm.at[pl.ds(0, CH)], rows[b], la[b]).wait()
        pltpu.make_async_copy(sg1_hbm.at[pl.ds(0, CH)], didx[b], di[b]).wait()
        pltpu.make_async_copy(sw_hbm.at[pl.ds(0, CH)], wv[b], ws[b]).wait()

    pref_b(0, 0)

    @pl.loop(0, n_b, step=2)
    def _(g):
        for b in range(2):
            i = g + b
            nb = 1 - b
            if b == 0:
                pref_b(i + 1, nb)
            else:
                @pl.when(g < n_b - 2)
                def _():
                    pref_b(i + 1, nb)
            drain_b(b)
            _scale_rows(rows[b], wv[b], CH)
            pltpu.sync_copy(rows[b], acc_a.at[didx[b]], add=True)

    plsc.subcore_barrier()
    pltpu.sync_copy(acc_n.at[pl.ds(s * 64, 64)], nmsg_out.at[c, pl.ds(s * 64, 64)])
    pltpu.sync_copy(acc_a.at[pl.ds(s * 64, 64)], amsg_out.at[c, pl.ds(s * 64, 64)])


N_ACC = 10112
CH2 = 64
_ED_W = N_EDGES // NW
_ED_FULL = _ED_W // CH
_ED_TAIL = _ED_W - _ED_FULL * CH
_ED2_FULL = _ED_W // CH2


@functools.cache
def _build_sc2():
    return functools.partial(
        pl.kernel,
        out_type=(jax.ShapeDtypeStruct((NC, N_ACC, D), _f32),
                  jax.ShapeDtypeStruct((NC, N_ACC, D), _f32)),
        mesh=_mesh(),
        scratch_types=[
            [pltpu.VMEM((CH2,), _i32)] * 2,
            [pltpu.VMEM((CH2,), _i32)] * 2,
            pltpu.VMEM((_ED_TAIL,), _i32),
            [pltpu.VMEM((CH2, L), _f32)] * 2,
            [pltpu.VMEM((CH2, D), _f32)] * 2,
            pltpu.VMEM_SHARED((N_ACC, D), _f32),
            pltpu.SemaphoreType.DMA,
            [pltpu.SemaphoreType.DMA] * 2,
            [pltpu.SemaphoreType.DMA] * 2,
            [pltpu.SemaphoreType.DMA] * 2,
            [pltpu.SemaphoreType.DMA] * 2,
        ],
    )(_sc2_body)


def _sc2_body(sn_hbm, b0_hbm, b1_hbm, bw_hbm, edges_hbm, g1_hbm,
              smsg_out, emsg_out, sidx, didx, didx_t, wv, rows, acc,
              sem, si, di, ws, la):
    c = lax.axis_index("c")
    s = lax.axis_index("s")
    wid = s * NC + c
    row0 = s * 632

    def zero_acc():
        _zero_vmem(rows[0], CH2)
        for t in range(9):
            pltpu.sync_copy(rows[0], acc.at[pl.ds(row0 + t * 64, 64)])
        pltpu.sync_copy(rows[0].at[pl.ds(0, 56)],
                        acc.at[pl.ds(row0 + 576, 56)])

    zero_acc()
    plsc.subcore_barrier()

    nb_w = NBP // NW
    base_b = wid * nb_w
    n_a = nb_w // CH2

    def pref_a(i, b):
        base = base_b + i * CH2
        pltpu.async_copy(b1_hbm.at[pl.ds(base, CH2)], sidx[b], si[b])
        pltpu.async_copy(b0_hbm.at[pl.ds(base, CH2)], didx[b], di[b])
        pltpu.async_copy(bw_hbm.at[pl.ds(base, CH2)], wv[b], ws[b])

    def drain_a(b):
        pltpu.make_async_copy(b1_hbm.at[pl.ds(0, CH2)], sidx[b], si[b]).wait()
        pltpu.make_async_copy(b0_hbm.at[pl.ds(0, CH2)], didx[b], di[b]).wait()
        pltpu.make_async_copy(bw_hbm.at[pl.ds(0, CH2)], wv[b], ws[b]).wait()

    pref_a(0, 0)

    @pl.loop(0, n_a, step=2)
    def _(g):
        for b in range(2):
            i = g + b
            nb = 1 - b
            if b == 0:
                pref_a(i + 1, nb)
            else:
                @pl.when(g < n_a - 2)
                def _():
                    pref_a(i + 1, nb)
            drain_a(b)
            pltpu.async_copy(sn_hbm.at[sidx[b]], rows[b], sem).wait()
            _scale_rows(rows[b], wv[b], CH2)
            pltpu.sync_copy(rows[b], acc.at[didx[b]], add=True)

    plsc.subcore_barrier()
    pltpu.sync_copy(acc.at[pl.ds(row0, 632)], smsg_out.at[c, pl.ds(row0, 632)])

    zero_acc()
    plsc.subcore_barrier()

    base_e = wid * _ED_W

    def pref_e(i, b):
        base = base_e + i * CH2
        pltpu.async_copy(edges_hbm.at[pl.ds(base, CH2)], rows[b], la[b])
        pltpu.async_copy(g1_hbm.at[pl.ds(base, CH2)], didx[b], di[b])

    def drain_e(b):
        pltpu.make_async_copy(edges_hbm.at[pl.ds(0, CH2)], rows[b],
                              la[b]).wait()
        pltpu.make_async_copy(g1_hbm.at[pl.ds(0, CH2)], didx[b],
                              di[b]).wait()

    pref_e(0, 0)

    @pl.loop(0, _ED2_FULL, step=2)
    def _(g):
        for b in range(2):
            i = g + b
            nb = 1 - b
            if b == 0:
                pref_e(i + 1, nb)
            else:
                @pl.when(g < _ED2_FULL - 2)
                def _():
                    pref_e(i + 1, nb)
            drain_e(b)
            pltpu.sync_copy(rows[b], acc.at[didx[b]], add=True)

    tbase = base_e + _ED2_FULL * CH2
    pltpu.sync_copy(edges_hbm.at[pl.ds(tbase, _ED_TAIL)],
                    rows[0].at[pl.ds(0, _ED_TAIL)])
    pltpu.sync_copy(g1_hbm.at[pl.ds(tbase, _ED_TAIL)], didx_t)
    pltpu.sync_copy(rows[0].at[pl.ds(0, _ED_TAIL)], acc.at[didx_t], add=True)

    plsc.subcore_barrier()
    pltpu.sync_copy(acc.at[pl.ds(row0, 632)], emsg_out.at[c, pl.ds(row0, 632)])


@functools.cache
def _build_sc34():
    return functools.partial(
        pl.kernel,
        out_type=(jax.ShapeDtypeStruct((NSP, D), _f32),
                  jax.ShapeDtypeStruct((N_EDGES, D), _f32),
                  jax.ShapeDtypeStruct((N_EDGES, D), _f32)),
        mesh=_mesh(),
        scratch_types=[
            pltpu.VMEM((_ED_W,), _i32),
            pltpu.VMEM((_ED_W,), _i32),
            [pltpu.VMEM((CH, D), _f32)] * 2,
            [pltpu.VMEM((CH, D), _f32)] * 2,
            [pltpu.SemaphoreType.DMA] * 2,
            [pltpu.SemaphoreType.DMA] * 2,
            [pltpu.SemaphoreType.DMA] * 2,
            [pltpu.SemaphoreType.DMA] * 2,
        ],
    )(_sc34_body)


def _sc34_body(p_hbm, q_hbm, sg0_hbm, sg1_hbm, nodes_hbm, g0_hbm, g1_hbm,
               t_out, na_out, nb_out, idx0, idx1, rowsa, rowsb, ga, gb,
               wa, wb):
    c = lax.axis_index("c")
    s = lax.axis_index("s")
    wid = s * NC + c

    ns_w = NSP // NW
    base_s = wid * ns_w
    n_t = ns_w // CH
    pltpu.sync_copy(sg0_hbm.at[pl.ds(base_s, ns_w)], idx0.at[pl.ds(0, ns_w)])
    pltpu.sync_copy(sg1_hbm.at[pl.ds(base_s, ns_w)], idx1.at[pl.ds(0, ns_w)])

    @pl.loop(0, n_t, step=2)
    def _(g):
        for b in range(2):
            i = g + b
            base = base_s + i * CH
            @pl.when(g > 0)
            def _():
                pltpu.make_async_copy(rowsa[b], t_out.at[pl.ds(0, CH)],
                                      wa[b]).wait()
            cpa = pltpu.async_copy(
                p_hbm.at[idx0.at[pl.ds(i * CH, CH)]], rowsa[b], ga[b])
            cpb = pltpu.async_copy(
                q_hbm.at[idx1.at[pl.ds(i * CH, CH)]], rowsb[b], gb[b])
            cpa.wait()
            cpb.wait()
            ra = rowsa[b]
            rb = rowsb[b]

            @pl.loop(0, CH)
            def _(r):
                for j in range(FC):
                    ra[r, pl.ds(j * L, L)] = (
                        ra[r, pl.ds(j * L, L)] + rb[r, pl.ds(j * L, L)])

            pltpu.async_copy(rowsa[b], t_out.at[pl.ds(base, CH)], wa[b])

    pltpu.make_async_copy(rowsa[0], t_out.at[pl.ds(0, CH)], wa[0]).wait()
    pltpu.make_async_copy(rowsa[1], t_out.at[pl.ds(0, CH)], wa[1]).wait()

    base_e = wid * _ED_W
    pltpu.sync_copy(g0_hbm.at[pl.ds(base_e, _ED_W)], idx0)
    pltpu.sync_copy(g1_hbm.at[pl.ds(base_e, _ED_W)], idx1)

    @pl.loop(0, _ED_FULL, step=2)
    def _(g):
        for b in range(2):
            i = g + b
            nb = 1 - b
            base = base_e + i * CH
            cpa = pltpu.async_copy(
                nodes_hbm.at[idx0.at[pl.ds(i * CH, CH)]], rowsa[b], ga[b])
            cpb = pltpu.async_copy(
                nodes_hbm.at[idx1.at[pl.ds(i * CH, CH)]], rowsb[b], gb[b])

            def drain_prev():
                pltpu.make_async_copy(rowsa[nb], na_out.at[pl.ds(0, CH)],
                                      wa[nb]).wait()
                pltpu.make_async_copy(rowsb[nb], nb_out.at[pl.ds(0, CH)],
                                      wb[nb]).wait()

            if b == 0:
                @pl.when(g > 0)
                def _():
                    drain_prev()
            else:
                drain_prev()
            cpa.wait()
            cpb.wait()
            pltpu.async_copy(rowsa[b], na_out.at[pl.ds(base, CH)], wa[b])
            pltpu.async_copy(rowsb[b], nb_out.at[pl.ds(base, CH)], wb[b])

    pltpu.make_async_copy(rowsa[1], na_out.at[pl.ds(0, CH)], wa[1]).wait()
    pltpu.make_async_copy(rowsb[1], nb_out.at[pl.ds(0, CH)], wb[1]).wait()

    tbase = base_e + _ED_FULL * CH
    pltpu.async_copy(nodes_hbm.at[idx0.at[pl.ds(_ED_FULL * CH, _ED_TAIL)]],
                     rowsa[0].at[pl.ds(0, _ED_TAIL)], ga[0]).wait()
    pltpu.async_copy(nodes_hbm.at[idx1.at[pl.ds(_ED_FULL * CH, _ED_TAIL)]],
                     rowsb[0].at[pl.ds(0, _ED_TAIL)], gb[0]).wait()
    pltpu.sync_copy(rowsa[0].at[pl.ds(0, _ED_TAIL)],
                    na_out.at[pl.ds(tbase, _ED_TAIL)])
    pltpu.sync_copy(rowsb[0].at[pl.ds(0, _ED_TAIL)],
                    nb_out.at[pl.ds(tbase, _ED_TAIL)])


def _dot(a, b):
    return jnp.dot(a, b, preferred_element_type=_f32)


def _sn_body(x, a0, a1, n0, n1, w1x, w1a, w1n, b1, w2, b2, pa, pb, bp,
             xo, po, qo):
    att = a0[0] + a1[0]
    nm = n0[0] + n1[0]
    h = jnp.maximum(
        _dot(x[...], w1x[...]) + _dot(att, w1a[...]) + _dot(nm, w1n[...])
        + b1[...], 0.0)
    xn = jnp.maximum(_dot(h, w2[...]) + b2[...], 0.0) + x[...]
    xo[...] = xn
    po[...] = _dot(xn, pa[...]) + bp[...]
    qo[...] = _dot(xn, pb[...])


def _nn_body(x, e0, e1, s0, s1, w1x, w1e, w1s, b1, w2, b2, xo):
    em = e0[0] + e1[0]
    sm = s0[0] + s1[0]
    h = jnp.maximum(
        _dot(x[...], w1x[...]) + _dot(em, w1e[...]) + _dot(sm, w1s[...])
        + b1[...], 0.0)
    xo[...] = jnp.maximum(_dot(h, w2[...]) + b2[...], 0.0) + x[...]


def _dotb(a, b):
    return jnp.dot(a, b, preferred_element_type=_f32)


def _se_body(t, e, c, w2, b2, out):
    h = jnp.maximum(t[...] + _dotb(e[...], c[...]), 0.0)
    out[...] = jnp.tanh(_dotb(h, w2[...]) + b2[...]) + e[...]


def _en_body(na, nb, e, w1a, w1b, w1c, b1, w2, b2, out):
    h = jnp.maximum(
        _dotb(na[...], w1a[...]) + _dotb(nb[...], w1b[...])
        + _dotb(e[...], w1c[...]) + b1[...], 0.0)
    out[...] = jnp.tanh(_dotb(h, w2[...]) + b2[...]) + e[...]


def _row_spec(rows):
    return pl.BlockSpec((rows, D), lambda i: (i, 0))


def _w_spec(shape):
    return pl.BlockSpec(shape, lambda i: tuple(0 for _ in shape))


def _part_spec(rows, core):
    return pl.BlockSpec((1, rows, D), lambda i, _c=core: (_c, i, 0))


def _tc_node_mlp(n, rows, x, msg1, msg2, w1, b1, w2, b2):
    grid = (n // rows,)
    ws = _w_spec((D, D))
    bs = _w_spec((1, D))
    ps0 = _part_spec(rows, 0)
    ps1 = _part_spec(rows, 1)
    return pl.pallas_call(
        _nn_body,
        grid=grid,
        in_specs=[_row_spec(rows), ps0, ps1, ps0, ps1, ws, ws, ws, bs, ws, bs],
        out_specs=_row_spec(rows),
        out_shape=jax.ShapeDtypeStruct((n, D), _f32),
        compiler_params=pltpu.CompilerParams(
            dimension_semantics=("arbitrary",)),
    )(x, msg1, msg1, msg2, msg2, w1[:D], w1[D:2 * D], w1[2 * D:],
      b1.reshape(1, D), w2, b2.reshape(1, D))


def _tc_sn_mlp(x, amsg, nmsg, w1, b1, w2, b2, pa, pb, bp):
    outs = [jax.ShapeDtypeStruct((N_SUPER, D), _f32)] * 3
    xs = pl.BlockSpec((N_SUPER, D), lambda i: (0, 0))
    ws = pl.BlockSpec((D, D), lambda i: (0, 0))
    bs = pl.BlockSpec((1, D), lambda i: (0, 0))
    ps0 = pl.BlockSpec((1, N_SUPER, D), lambda i: (0, 0, 0))
    ps1 = pl.BlockSpec((1, N_SUPER, D), lambda i: (1, 0, 0))
    return pl.pallas_call(
        _sn_body,
        grid=(1,),
        in_specs=[xs, ps0, ps1, ps0, ps1, ws, ws, ws, bs, ws, bs, ws, ws, bs],
        out_specs=[xs] * 3,
        out_shape=outs,
    )(x, amsg, amsg, nmsg, nmsg, w1[:D], w1[D:2 * D], w1[2 * D:],
      b1.reshape(1, D), w2, b2.reshape(1, D), pa, pb, bp.reshape(1, D))


def _tc_se_mlp(t, e, c, w2, b2):
    rows = 2000
    grid = (N_SED // rows,)
    ws = _w_spec((D, D))
    bs = _w_spec((1, D))
    return pl.pallas_call(
        _se_body,
        grid=grid,
        in_specs=[_row_spec(rows), _row_spec(rows), ws, ws, bs],
        out_specs=_row_spec(rows),
        out_shape=jax.ShapeDtypeStruct((N_SED, D), _f32),
        compiler_params=pltpu.CompilerParams(
            dimension_semantics=("arbitrary",)),
    )(t, e, c, w2, b2.reshape(1, D))


def _tc_en_mlp(na, nb, e, w1, b1, w2, b2):
    rows = 2000
    grid = (N_EDGES // rows,)
    ws = _w_spec((D, D))
    bs = _w_spec((1, D))
    return pl.pallas_call(
        _en_body,
        grid=grid,
        in_specs=[_row_spec(rows)] * 3 + [ws, ws, ws, bs, ws, bs],
        out_specs=_row_spec(rows),
        out_shape=jax.ShapeDtypeStruct((N_EDGES, D), _f32),
        compiler_params=pltpu.CompilerParams(
            dimension_semantics=("arbitrary",)),
    )(na, nb, e, w1[:D], w1[D:2 * D], w1[2 * D:], b1.reshape(1, D),
      w2, b2.reshape(1, D))


def kernel(nodes, edges, supernodes, superedges, graph, bipartite_graph,
           bipartite_edge_weights, super_graph, super_edge_weights,
           en_W1, en_b1, en_W2, en_b2, nn_W1, nn_b1, nn_W2, nn_b2,
           sn_W1, sn_b1, sn_W2, sn_b2, se_W1, se_b1, se_W2, se_b2):
    g0 = graph[0]
    g1 = graph[1]
    b0 = jnp.pad(bipartite_graph[0], (0, NBP - N_BIP))
    b1i = jnp.pad(bipartite_graph[1], (0, NBP - N_BIP))
    bw = jnp.broadcast_to(
        jnp.pad(bipartite_edge_weights, ((0, NBP - N_BIP), (0, 0))), (NBP, L))
    sedp = jnp.pad(superedges, ((0, NSP - N_SED), (0, 0)))
    sw = jnp.broadcast_to(
        jnp.pad(super_edge_weights, ((0, NSP - N_SED), (0, 0))), (NSP, L))
    sg0 = jnp.pad(super_graph[0], (0, NSP - N_SED))
    sg1 = jnp.pad(super_graph[1], (0, NSP - N_SED))

    nmsg, amsg = _build_sc1()(nodes, b0, b1i, bw, sedp, sw, sg1)

    sn_new, p_se, q_se = _tc_sn_mlp(
        supernodes, amsg, nmsg, sn_W1, sn_b1, sn_W2, sn_b2,
        se_W1[:D], se_W1[D:2 * D], se_b1)

    smsg, emsg = _build_sc2()(sn_new, b0, b1i, bw, edges, g1)

    nodes_new = _tc_node_mlp(
        N_NODES, 1000, nodes, emsg, smsg, nn_W1, nn_b1, nn_W2, nn_b2)

    t_se, na, nb = _build_sc34()(p_se, q_se, sg0, sg1, nodes_new, g0, g1)
    sed_new = _tc_se_mlp(t_se, superedges, se_W1[2 * D:], se_W2, se_b2)
    edges_new = _tc_en_mlp(na, nb, edges, en_W1, en_b1, en_W2, en_b2)

    return (nodes_new, edges_new, sn_new, sed_new)


# speedup vs baseline: 1.0560x; 1.0189x over previous
import functools

import jax
import jax.numpy as jnp
from jax import lax
from jax.experimental import pallas as pl
from jax.experimental.pallas import tpu as pltpu
from jax.experimental.pallas import tpu_sc as plsc

D = 128
L = 16
FC = D // L
NC = 2
NSUB = 16
NW = NC * NSUB

N_NODES = 10000
N_EDGES = 320000
N_SUPER = 1000
N_BIP = 40000
N_SED = 16000
NBP = 40960
NSP = 16384
CH = 128

@functools.cache
def _mesh():
    return plsc.VectorSubcoreMesh(
        core_axis_name="c", subcore_axis_name="s",
        num_cores=NC, num_subcores=NSUB)


_f32 = jnp.float32
_i32 = jnp.int32


def _zero_vmem(buf, nrows):
    z = jnp.zeros((L,), _f32)

    @pl.loop(0, nrows)
    def _(r):
        for j in range(FC):
            buf[r, pl.ds(j * L, L)] = z


def _scale_rows(rows, wv, nrows):
    @pl.loop(0, nrows)
    def _(r):
        wr = wv[r, :]
        for j in range(FC):
            rows[r, pl.ds(j * L, L)] = rows[r, pl.ds(j * L, L)] * wr


@functools.cache
def _build_sc1():
    return functools.partial(
        pl.kernel,
        out_type=(jax.ShapeDtypeStruct((NC, 1024, D), _f32),
                  jax.ShapeDtypeStruct((NC, 1024, D), _f32)),
        mesh=_mesh(),
        scratch_types=[
            [pltpu.VMEM((CH,), _i32)] * 2,
            [pltpu.VMEM((CH,), _i32)] * 2,
            [pltpu.VMEM((CH, L), _f32)] * 2,
            [pltpu.VMEM((CH, D), _f32)] * 2,
            pltpu.VMEM((64, D), _f32),
            pltpu.VMEM_SHARED((1024, D), _f32),
            pltpu.VMEM_SHARED((1024, D), _f32),
            pltpu.SemaphoreType.DMA,
            [pltpu.SemaphoreType.DMA] * 2,
            [pltpu.SemaphoreType.DMA] * 2,
            [pltpu.SemaphoreType.DMA] * 2,
            [pltpu.SemaphoreType.DMA] * 2,
        ],
    )(_sc1_body)


def _sc1_body(nodes_hbm, b0_hbm, b1_hbm, bw_hbm, sed_hbm, sw_hbm, sg1_hbm,
              nmsg_out, amsg_out, sidx, didx, wv, rows, zbuf, acc_n, acc_a,
              sem, si, di, ws, la):
    c = lax.axis_index("c")
    s = lax.axis_index("s")
    wid = s * NC + c

    _zero_vmem(zbuf, 64)
    pltpu.sync_copy(zbuf, acc_n.at[pl.ds(s * 64, 64)])
    pltpu.sync_copy(zbuf, acc_a.at[pl.ds(s * 64, 64)])
    plsc.subcore_barrier()

    nb_w = NBP // NW
    base_b = wid * nb_w
    n_a = nb_w // CH

    def pref_a(i, b):
        base = base_b + i * CH
        pltpu.async_copy(b0_hbm.at[pl.ds(base, CH)], sidx[b], si[b])
        pltpu.async_copy(b1_hbm.at[pl.ds(base, CH)], didx[b], di[b])
        pltpu.async_copy(bw_hbm.at[pl.ds(base, CH)], wv[b], ws[b])

    def drain_a(b):
        pltpu.make_async_copy(b0_hbm.at[pl.ds(0, CH)], sidx[b], si[b]).wait()
        pltpu.make_async_copy(b1_hbm.at[pl.ds(0, CH)], didx[b], di[b]).wait()
        pltpu.make_async_copy(bw_hbm.at[pl.ds(0, CH)], wv[b], ws[b]).wait()

    pref_a(0, 0)

    @pl.loop(0, n_a, step=2)
    def _(g):
        for b in range(2):
            i = g + b
            nb = 1 - b
            if b == 0:
                pref_a(i + 1, nb)
            else:
                @pl.when(g < n_a - 2)
                def _():
                    pref_a(i + 1, nb)
            drain_a(b)
            pltpu.async_copy(nodes_hbm.at[sidx[b]], rows[b], sem).wait()
            _scale_rows(rows[b], wv[b], CH)
            pltpu.sync_copy(rows[b], acc_n.at[didx[b]], add=True)

    ns_w = NSP // NW
    base_s = wid * ns_w
    n_b = ns_w // CH

    def pref_b(i, b):
        base = base_s + i * CH
        pltpu.async_copy(sed_hbm.at[pl.ds(base, CH)], rows[b], la[b])
        pltpu.async_copy(sg1_hbm.at[pl.ds(base, CH)], didx[b], di[b])
        pltpu.async_copy(sw_hbm.at[pl.ds(base, CH)], wv[b], ws[b])

    def drain_b(b):
        pltpu.make_async_copy(sed_hbm.at[pl.ds(0, CH)], rows[b], la[b]).wait()
        pltpu.make_async_copy(sg1_hbm.at[pl.ds(0, CH)], didx[b], di[b]).wait()
        pltpu.make_async_copy(sw_hbm.at[pl.ds(0, CH)], wv[b], ws[b]).wait()

    pref_b(0, 0)

    @pl.loop(0, n_b, step=2)
    def _(g):
        for b in range(2):
            i = g + b
            nb = 1 - b
            if b == 0:
                pref_b(i + 1, nb)
            else:
                @pl.when(g < n_b - 2)
                def _():
                    pref_b(i + 1, nb)
            drain_b(b)
            _scale_rows(rows[b], wv[b], CH)
            pltpu.sync_copy(rows[b], acc_a.at[didx[b]], add=True)

    plsc.subcore_barrier()
    pltpu.sync_copy(acc_n.at[pl.ds(s * 64, 64)], nmsg_out.at[c, pl.ds(s * 64, 64)])
    pltpu.sync_copy(acc_a.at[pl.ds(s * 64, 64)], amsg_out.at[c, pl.ds(s * 64, 64)])


N_ACC = 10112
CH2 = 64
_ED_W = N_EDGES // NW
_ED_FULL = _ED_W // CH
_ED_TAIL = _ED_W - _ED_FULL * CH
_ED2_FULL = _ED_W // CH2


@functools.cache
def _build_sc2():
    return functools.partial(
        pl.kernel,
        out_type=(jax.ShapeDtypeStruct((NC, N_ACC, D), _f32),
                  jax.ShapeDtypeStruct((NC, N_ACC, D), _f32),
                  jax.ShapeDtypeStruct((NSP, D), _f32)),
        mesh=_mesh(),
        scratch_types=[
            [pltpu.VMEM((CH2,), _i32)] * 2,
            [pltpu.VMEM((CH2,), _i32)] * 2,
            pltpu.VMEM((_ED_TAIL,), _i32),
            [pltpu.VMEM((CH2, L), _f32)] * 2,
            [pltpu.VMEM((CH2, D), _f32)] * 2,
            pltpu.VMEM_SHARED((N_ACC, D), _f32),
            pltpu.SemaphoreType.DMA,
            [pltpu.SemaphoreType.DMA] * 2,
            [pltpu.SemaphoreType.DMA] * 2,
            [pltpu.SemaphoreType.DMA] * 2,
            [pltpu.SemaphoreType.DMA] * 2,
        ],
    )(_sc2_body)


def _sc2_body(sn_hbm, b0_hbm, b1_hbm, bw_hbm, edges_hbm, g1_hbm,
              p_hbm, q_hbm, sg0_hbm, sg1_hbm,
              smsg_out, emsg_out, t_out, sidx, didx, didx_t, wv, rows, acc,
              sem, si, di, ws, la):
    c = lax.axis_index("c")
    s = lax.axis_index("s")
    wid = s * NC + c
    row0 = s * 632

    def zero_acc():
        _zero_vmem(rows[0], CH2)
        for t in range(9):
            pltpu.sync_copy(rows[0], acc.at[pl.ds(row0 + t * 64, 64)])
        pltpu.sync_copy(rows[0].at[pl.ds(0, 56)],
                        acc.at[pl.ds(row0 + 576, 56)])

    zero_acc()

    ns_w = NSP // NW
    base_t = wid * ns_w

    @pl.loop(0, ns_w // CH2)
    def _(i):
        base = base_t + i * CH2
        pltpu.sync_copy(sg0_hbm.at[pl.ds(base, CH2)], sidx[0])
        pltpu.sync_copy(sg1_hbm.at[pl.ds(base, CH2)], sidx[1])
        cpa = pltpu.async_copy(p_hbm.at[sidx[0]], rows[0], sem)
        cpb = pltpu.async_copy(q_hbm.at[sidx[1]], rows[1], la[0])
        cpa.wait()
        cpb.wait()

        @pl.loop(0, CH2)
        def _(r):
            for j in range(FC):
                rows[0][r, pl.ds(j * L, L)] = (
                    rows[0][r, pl.ds(j * L, L)]
                    + rows[1][r, pl.ds(j * L, L)])

        pltpu.sync_copy(rows[0], t_out.at[pl.ds(base, CH2)])

    plsc.subcore_barrier()

    nb_w = NBP // NW
    base_b = wid * nb_w
    n_a = nb_w // CH2

    def pref_a(i, b):
        base = base_b + i * CH2
        pltpu.async_copy(b1_hbm.at[pl.ds(base, CH2)], sidx[b], si[b])
        pltpu.async_copy(b0_hbm.at[pl.ds(base, CH2)], didx[b], di[b])
        pltpu.async_copy(bw_hbm.at[pl.ds(base, CH2)], wv[b], ws[b])

    def drain_a(b):
        pltpu.make_async_copy(b1_hbm.at[pl.ds(0, CH2)], sidx[b], si[b]).wait()
        pltpu.make_async_copy(b0_hbm.at[pl.ds(0, CH2)], didx[b], di[b]).wait()
        pltpu.make_async_copy(bw_hbm.at[pl.ds(0, CH2)], wv[b], ws[b]).wait()

    pref_a(0, 0)

    @pl.loop(0, n_a, step=2)
    def _(g):
        for b in range(2):
            i = g + b
            nb = 1 - b
            if b == 0:
                pref_a(i + 1, nb)
            else:
                @pl.when(g < n_a - 2)
                def _():
                    pref_a(i + 1, nb)
            drain_a(b)
            pltpu.async_copy(sn_hbm.at[sidx[b]], rows[b], sem).wait()
            _scale_rows(rows[b], wv[b], CH2)
            pltpu.sync_copy(rows[b], acc.at[didx[b]], add=True)

    plsc.subcore_barrier()
    pltpu.sync_copy(acc.at[pl.ds(row0, 632)], smsg_out.at[c, pl.ds(row0, 632)])

    zero_acc()
    plsc.subcore_barrier()

    base_e = wid * _ED_W

    def pref_e(i, b):
        base = base_e + i * CH2
        pltpu.async_copy(edges_hbm.at[pl.ds(base, CH2)], rows[b], la[b])
        pltpu.async_copy(g1_hbm.at[pl.ds(base, CH2)], didx[b], di[b])

    def drain_e(b):
        pltpu.make_async_copy(edges_hbm.at[pl.ds(0, CH2)], rows[b],
                              la[b]).wait()
        pltpu.make_async_copy(g1_hbm.at[pl.ds(0, CH2)], didx[b],
                              di[b]).wait()

    pref_e(0, 0)

    @pl.loop(0, _ED2_FULL, step=2)
    def _(g):
        for b in range(2):
            i = g + b
            nb = 1 - b
            if b == 0:
                pref_e(i + 1, nb)
            else:
                @pl.when(g < _ED2_FULL - 2)
                def _():
                    pref_e(i + 1, nb)
            drain_e(b)
            pltpu.sync_copy(rows[b], acc.at[didx[b]], add=True)

    tbase = base_e + _ED2_FULL * CH2
    pltpu.sync_copy(edges_hbm.at[pl.ds(tbase, _ED_TAIL)],
                    rows[0].at[pl.ds(0, _ED_TAIL)])
    pltpu.sync_copy(g1_hbm.at[pl.ds(tbase, _ED_TAIL)], didx_t)
    pltpu.sync_copy(rows[0].at[pl.ds(0, _ED_TAIL)], acc.at[didx_t], add=True)

    plsc.subcore_barrier()
    pltpu.sync_copy(acc.at[pl.ds(row0, 632)], emsg_out.at[c, pl.ds(row0, 632)])


@functools.cache
def _build_sc4():
    return functools.partial(
        pl.kernel,
        out_type=(jax.ShapeDtypeStruct((N_EDGES, D), _f32),
                  jax.ShapeDtypeStruct((N_EDGES, D), _f32)),
        mesh=_mesh(),
        scratch_types=[
            pltpu.VMEM((_ED_W,), _i32),
            pltpu.VMEM((_ED_W,), _i32),
            [pltpu.VMEM((CH, D), _f32)] * 3,
            [pltpu.VMEM((CH, D), _f32)] * 3,
            [pltpu.SemaphoreType.DMA] * 3,
            [pltpu.SemaphoreType.DMA] * 3,
            [pltpu.SemaphoreType.DMA] * 3,
            [pltpu.SemaphoreType.DMA] * 3,
        ],
    )(_sc4_body)


def _sc4_body(nodes_hbm, g0_hbm, g1_hbm, na_out, nb_out,
              idx0, idx1, rowsa, rowsb, ga, gb, wa, wb):
    c = lax.axis_index("c")
    s = lax.axis_index("s")
    wid = s * NC + c
    base_e = wid * _ED_W

    pltpu.sync_copy(g0_hbm.at[pl.ds(base_e, _ED_W)], idx0)
    pltpu.sync_copy(g1_hbm.at[pl.ds(base_e, _ED_W)], idx1)

    @pl.loop(0, _ED_FULL, step=3)
    def _(g):
        for b in range(3):
            i = g + b
            db = (b + 1) % 3
            base = base_e + i * CH
            cpa = pltpu.async_copy(
                nodes_hbm.at[idx0.at[pl.ds(i * CH, CH)]], rowsa[b], ga[b])
            cpb = pltpu.async_copy(
                nodes_hbm.at[idx1.at[pl.ds(i * CH, CH)]], rowsb[b], gb[b])

            def drain_prev():
                pltpu.make_async_copy(rowsa[db], na_out.at[pl.ds(0, CH)],
                                      wa[db]).wait()
                pltpu.make_async_copy(rowsb[db], nb_out.at[pl.ds(0, CH)],
                                      wb[db]).wait()

            if b < 2:
                @pl.when(g > 0)
                def _():
                    drain_prev()
            else:
                drain_prev()
            cpa.wait()
            cpb.wait()
            pltpu.async_copy(rowsa[b], na_out.at[pl.ds(base, CH)], wa[b])
            pltpu.async_copy(rowsb[b], nb_out.at[pl.ds(base, CH)], wb[b])

    for b in (1, 2):
        pltpu.make_async_copy(rowsa[b], na_out.at[pl.ds(0, CH)], wa[b]).wait()
        pltpu.make_async_copy(rowsb[b], nb_out.at[pl.ds(0, CH)], wb[b]).wait()

    tbase = base_e + _ED_FULL * CH
    pltpu.async_copy(nodes_hbm.at[idx0.at[pl.ds(_ED_FULL * CH, _ED_TAIL)]],
                     rowsa[0].at[pl.ds(0, _ED_TAIL)], ga[0]).wait()
    pltpu.async_copy(nodes_hbm.at[idx1.at[pl.ds(_ED_FULL * CH, _ED_TAIL)]],
                     rowsb[0].at[pl.ds(0, _ED_TAIL)], gb[0]).wait()
    pltpu.sync_copy(rowsa[0].at[pl.ds(0, _ED_TAIL)],
                    na_out.at[pl.ds(tbase, _ED_TAIL)])
    pltpu.sync_copy(rowsb[0].at[pl.ds(0, _ED_TAIL)],
                    nb_out.at[pl.ds(tbase, _ED_TAIL)])


def _dot(a, b):
    return jnp.dot(a, b, preferred_element_type=_f32)


def _sn_body(x, a0, a1, n0, n1, w1x, w1a, w1n, b1, w2, b2, pa, pb, bp,
             xo, po, qo):
    att = a0[0] + a1[0]
    nm = n0[0] + n1[0]
    h = jnp.maximum(
        _dot(x[...], w1x[...]) + _dot(att, w1a[...]) + _dot(nm, w1n[...])
        + b1[...], 0.0)
    xn = jnp.maximum(_dot(h, w2[...]) + b2[...], 0.0) + x[...]
    xo[...] = xn
    po[...] = _dot(xn, pa[...]) + bp[...]
    qo[...] = _dot(xn, pb[...])


def _nn_body(x, e0, e1, s0, s1, w1x, w1e, w1s, b1, w2, b2, xo):
    em = e0[0] + e1[0]
    sm = s0[0] + s1[0]
    h = jnp.maximum(
        _dot(x[...], w1x[...]) + _dot(em, w1e[...]) + _dot(sm, w1s[...])
        + b1[...], 0.0)
    xo[...] = jnp.maximum(_dot(h, w2[...]) + b2[...], 0.0) + x[...]


def _dotb(a, b):
    return jnp.dot(a, b, preferred_element_type=_f32)


def _se_body(t, e, c, w2, b2, out):
    h = jnp.maximum(t[...] + _dotb(e[...], c[...]), 0.0)
    out[...] = jnp.tanh(_dotb(h, w2[...]) + b2[...]) + e[...]


def _en_body(na, nb, e, w1a, w1b, w1c, b1, w2, b2, out):
    h = jnp.maximum(
        _dotb(na[...], w1a[...]) + _dotb(nb[...], w1b[...])
        + _dotb(e[...], w1c[...]) + b1[...], 0.0)
    out[...] = jnp.tanh(_dotb(h, w2[...]) + b2[...]) + e[...]


def _row_spec(rows):
    return pl.BlockSpec((rows, D), lambda i: (i, 0))


def _w_spec(shape):
    return pl.BlockSpec(shape, lambda i: tuple(0 for _ in shape))


def _part_spec(rows, core):
    return pl.BlockSpec((1, rows, D), lambda i, _c=core: (_c, i, 0))


def _tc_node_mlp(n, rows, x, msg1, msg2, w1, b1, w2, b2):
    grid = (n // rows,)
    ws = _w_spec((D, D))
    bs = _w_spec((1, D))
    ps0 = _part_spec(rows, 0)
    ps1 = _part_spec(rows, 1)
    return pl.pallas_call(
        _nn_body,
        grid=grid,
        in_specs=[_row_spec(rows), ps0, ps1, ps0, ps1, ws, ws, ws, bs, ws, bs],
        out_specs=_row_spec(rows),
        out_shape=jax.ShapeDtypeStruct((n, D), _f32),
        compiler_params=pltpu.CompilerParams(
            dimension_semantics=("arbitrary",)),
    )(x, msg1, msg1, msg2, msg2, w1[:D], w1[D:2 * D], w1[2 * D:],
      b1.reshape(1, D), w2, b2.reshape(1, D))


def _tc_sn_mlp(x, amsg, nmsg, w1, b1, w2, b2, pa, pb, bp):
    outs = [jax.ShapeDtypeStruct((N_SUPER, D), _f32)] * 3
    xs = pl.BlockSpec((N_SUPER, D), lambda i: (0, 0))
    ws = pl.BlockSpec((D, D), lambda i: (0, 0))
    bs = pl.BlockSpec((1, D), lambda i: (0, 0))
    ps0 = pl.BlockSpec((1, N_SUPER, D), lambda i: (0, 0, 0))
    ps1 = pl.BlockSpec((1, N_SUPER, D), lambda i: (1, 0, 0))
    return pl.pallas_call(
        _sn_body,
        grid=(1,),
        in_specs=[xs, ps0, ps1, ps0, ps1, ws, ws, ws, bs, ws, bs, ws, ws, bs],
        out_specs=[xs] * 3,
        out_shape=outs,
    )(x, amsg, amsg, nmsg, nmsg, w1[:D], w1[D:2 * D], w1[2 * D:],
      b1.reshape(1, D), w2, b2.reshape(1, D), pa, pb, bp.reshape(1, D))


def _tc_se_mlp(t, e, c, w2, b2):
    rows = 2000
    grid = (N_SED // rows,)
    ws = _w_spec((D, D))
    bs = _w_spec((1, D))
    return pl.pallas_call(
        _se_body,
        grid=grid,
        in_specs=[_row_spec(rows), _row_spec(rows), ws, ws, bs],
        out_specs=_row_spec(rows),
        out_shape=jax.ShapeDtypeStruct((N_SED, D), _f32),
        compiler_params=pltpu.CompilerParams(
            dimension_semantics=("arbitrary",)),
    )(t, e, c, w2, b2.reshape(1, D))


def _tc_en_mlp(na, nb, e, w1, b1, w2, b2):
    rows = 2000
    grid = (N_EDGES // rows,)
    ws = _w_spec((D, D))
    bs = _w_spec((1, D))
    return pl.pallas_call(
        _en_body,
        grid=grid,
        in_specs=[_row_spec(rows)] * 3 + [ws, ws, ws, bs, ws, bs],
        out_specs=_row_spec(rows),
        out_shape=jax.ShapeDtypeStruct((N_EDGES, D), _f32),
        compiler_params=pltpu.CompilerParams(
            dimension_semantics=("arbitrary",)),
    )(na, nb, e, w1[:D], w1[D:2 * D], w1[2 * D:], b1.reshape(1, D),
      w2, b2.reshape(1, D))


def kernel(nodes, edges, supernodes, superedges, graph, bipartite_graph,
           bipartite_edge_weights, super_graph, super_edge_weights,
           en_W1, en_b1, en_W2, en_b2, nn_W1, nn_b1, nn_W2, nn_b2,
           sn_W1, sn_b1, sn_W2, sn_b2, se_W1, se_b1, se_W2, se_b2):
    g0 = graph[0]
    g1 = graph[1]
    b0 = jnp.pad(bipartite_graph[0], (0, NBP - N_BIP))
    b1i = jnp.pad(bipartite_graph[1], (0, NBP - N_BIP))
    bw = jnp.broadcast_to(
        jnp.pad(bipartite_edge_weights, ((0, NBP - N_BIP), (0, 0))), (NBP, L))
    sedp = jnp.pad(superedges, ((0, NSP - N_SED), (0, 0)))
    sw = jnp.broadcast_to(
        jnp.pad(super_edge_weights, ((0, NSP - N_SED), (0, 0))), (NSP, L))
    sg0 = jnp.pad(super_graph[0], (0, NSP - N_SED))
    sg1 = jnp.pad(super_graph[1], (0, NSP - N_SED))

    nmsg, amsg = _build_sc1()(nodes, b0, b1i, bw, sedp, sw, sg1)

    sn_new, p_se, q_se = _tc_sn_mlp(
        supernodes, amsg, nmsg, sn_W1, sn_b1, sn_W2, sn_b2,
        se_W1[:D], se_W1[D:2 * D], se_b1)

    smsg, emsg, t_se = _build_sc2()(
        sn_new, b0, b1i, bw, edges, g1, p_se, q_se, sg0, sg1)

    nodes_new = _tc_node_mlp(
        N_NODES, 1000, nodes, emsg, smsg, nn_W1, nn_b1, nn_W2, nn_b2)

    sed_new = _tc_se_mlp(t_se, superedges, se_W1[2 * D:], se_W2, se_b2)
    na, nb = _build_sc4()(nodes_new, g0, g1)
    edges_new = _tc_en_mlp(na, nb, edges, en_W1, en_b1, en_W2, en_b2)

    return (nodes_new, edges_new, sn_new, sed_new)


# speedup vs baseline: 1.1341x; 1.0740x over previous
import functools

import jax
import jax.numpy as jnp
from jax import lax
from jax.experimental import pallas as pl
from jax.experimental.pallas import tpu as pltpu
from jax.experimental.pallas import tpu_sc as plsc

D = 128
L = 16
FC = D // L
NC = 2
NSUB = 16
NW = NC * NSUB

N_NODES = 10000
N_EDGES = 320000
N_SUPER = 1000
N_BIP = 40000
N_SED = 16000
NBP = 40960
NSP = 16384
CH = 128

@functools.cache
def _mesh():
    return plsc.VectorSubcoreMesh(
        core_axis_name="c", subcore_axis_name="s",
        num_cores=NC, num_subcores=NSUB)


_f32 = jnp.float32
_i32 = jnp.int32


def _zero_vmem(buf, nrows):
    z = jnp.zeros((L,), _f32)

    @pl.loop(0, nrows)
    def _(r):
        for j in range(FC):
            buf[r, pl.ds(j * L, L)] = z


def _scale_rows(rows, wv, nrows):
    @pl.loop(0, nrows)
    def _(r):
        wr = wv[r, :]
        for j in range(FC):
            rows[r, pl.ds(j * L, L)] = rows[r, pl.ds(j * L, L)] * wr


@functools.cache
def _build_sc1():
    return functools.partial(
        pl.kernel,
        out_type=(jax.ShapeDtypeStruct((NC, 1024, D), _f32),
                  jax.ShapeDtypeStruct((NC, 1024, D), _f32)),
        mesh=_mesh(),
        scratch_types=[
            [pltpu.VMEM((CH,), _i32)] * 2,
            [pltpu.VMEM((CH,), _i32)] * 2,
            [pltpu.VMEM((CH, L), _f32)] * 2,
            [pltpu.VMEM((CH, D), _f32)] * 2,
            pltpu.VMEM((64, D), _f32),
            pltpu.VMEM_SHARED((1024, D), _f32),
            pltpu.VMEM_SHARED((1024, D), _f32),
            pltpu.SemaphoreType.DMA,
            [pltpu.SemaphoreType.DMA] * 2,
            [pltpu.SemaphoreType.DMA] * 2,
            [pltpu.SemaphoreType.DMA] * 2,
            [pltpu.SemaphoreType.DMA] * 2,
        ],
    )(_sc1_body)


def _sc1_body(nodes_hbm, b0_hbm, b1_hbm, bw_hbm, sed_hbm, sw_hbm, sg1_hbm,
              nmsg_out, amsg_out, sidx, didx, wv, rows, zbuf, acc_n, acc_a,
              sem, si, di, ws, la):
    c = lax.axis_index("c")
    s = lax.axis_index("s")
    wid = s * NC + c

    _zero_vmem(zbuf, 64)
    pltpu.sync_copy(zbuf, acc_n.at[pl.ds(s * 64, 64)])
    pltpu.sync_copy(zbuf, acc_a.at[pl.ds(s * 64, 64)])
    plsc.subcore_barrier()

    nb_w = NBP // NW
    base_b = wid * nb_w
    n_a = nb_w // CH

    def pref_a(i, b):
        base = base_b + i * CH
        pltpu.async_copy(b0_hbm.at[pl.ds(base, CH)], sidx[b], si[b])
        pltpu.async_copy(b1_hbm.at[pl.ds(base, CH)], didx[b], di[b])
        pltpu.async_copy(bw_hbm.at[pl.ds(base, CH)], wv[b], ws[b])

    def drain_a(b):
        pltpu.make_async_copy(b0_hbm.at[pl.ds(0, CH)], sidx[b], si[b]).wait()
        pltpu.make_async_copy(b1_hbm.at[pl.ds(0, CH)], didx[b], di[b]).wait()
        pltpu.make_async_copy(bw_hbm.at[pl.ds(0, CH)], wv[b], ws[b]).wait()

    pref_a(0, 0)

    @pl.loop(0, n_a, step=2)
    def _(g):
        for b in range(2):
            i = g + b
            nb = 1 - b
            if b == 0:
                pref_a(i + 1, nb)
            else:
                @pl.when(g < n_a - 2)
                def _():
                    pref_a(i + 1, nb)
            drain_a(b)
            pltpu.async_copy(nodes_hbm.at[sidx[b]], rows[b], sem).wait()
            _scale_rows(rows[b], wv[b], CH)
            pltpu.sync_copy(rows[b], acc_n.at[didx[b]], add=True)

    ns_w = NSP // NW
    base_s = wid * ns_w
    n_b = ns_w // CH

    def pref_b(i, b):
        base = base_s + i * CH
        pltpu.async_copy(sed_hbm.at[pl.ds(base, CH)], rows[b], la[b])
        pltpu.async_copy(sg1_hbm.at[pl.ds(base, CH)], didx[b], di[b])
        pltpu.async_copy(sw_hbm.at[pl.ds(base, CH)], wv[b], ws[b])

    def drain_b(b):
        pltpu.make_async_copy(sed_hbm.at[pl.ds(0, CH)], rows[b], la[b]).wait()
        pltpu.make_async_copy(sg1_hbm.at[pl.ds(0, CH)], didx[b], di[b]).wait()
        pltpu.make_async_copy(sw_hbm.at[pl.ds(0, CH)], wv[b], ws[b]).wait()

    pref_b(0, 0)

    @pl.loop(0, n_b, step=2)
    def _(g):
        for b in range(2):
            i = g + b
            nb = 1 - b
            if b == 0:
                pref_b(i + 1, nb)
            else:
                @pl.when(g < n_b - 2)
                def _():
                    pref_b(i + 1, nb)
            drain_b(b)
            _scale_rows(rows[b], wv[b], CH)
            pltpu.sync_copy(rows[b], acc_a.at[didx[b]], add=True)

    plsc.subcore_barrier()
    pltpu.sync_copy(acc_n.at[pl.ds(s * 64, 64)], nmsg_out.at[c, pl.ds(s * 64, 64)])
    pltpu.sync_copy(acc_a.at[pl.ds(s * 64, 64)], amsg_out.at[c, pl.ds(s * 64, 64)])


N_ACC = 10112
CH2 = 64
_ED_W = N_EDGES // NW
_ED_FULL = _ED_W // CH
_ED_TAIL = _ED_W - _ED_FULL * CH
_ED2_FULL = _ED_W // CH2


@functools.cache
def _build_sc2():
    return functools.partial(
        pl.kernel,
        out_type=(jax.ShapeDtypeStruct((NC, N_ACC, D), _f32),
                  jax.ShapeDtypeStruct((NC, N_ACC, D), _f32),
                  jax.ShapeDtypeStruct((NSP, D), _f32)),
        mesh=_mesh(),
        scratch_types=[
            [pltpu.VMEM((CH2,), _i32)] * 2,
            [pltpu.VMEM((CH2,), _i32)] * 2,
            pltpu.VMEM((_ED_TAIL,), _i32),
            [pltpu.VMEM((CH2, L), _f32)] * 2,
            [pltpu.VMEM((CH2, D), _f32)] * 2,
            pltpu.VMEM_SHARED((N_ACC, D), _f32),
            pltpu.SemaphoreType.DMA,
            [pltpu.SemaphoreType.DMA] * 2,
            [pltpu.SemaphoreType.DMA] * 2,
            [pltpu.SemaphoreType.DMA] * 2,
            [pltpu.SemaphoreType.DMA] * 2,
        ],
    )(_sc2_body)


def _sc2_body(sn_hbm, b0_hbm, b1_hbm, bw_hbm, edges_hbm, g1_hbm,
              p_hbm, q_hbm, sg0_hbm, sg1_hbm,
              smsg_out, emsg_out, t_out, sidx, didx, didx_t, wv, rows, acc,
              sem, si, di, ws, la):
    c = lax.axis_index("c")
    s = lax.axis_index("s")
    wid = s * NC + c
    row0 = s * 632

    def zero_acc():
        _zero_vmem(rows[0], CH2)
        for t in range(9):
            pltpu.sync_copy(rows[0], acc.at[pl.ds(row0 + t * 64, 64)])
        pltpu.sync_copy(rows[0].at[pl.ds(0, 56)],
                        acc.at[pl.ds(row0 + 576, 56)])

    zero_acc()

    ns_w = NSP // NW
    base_t = wid * ns_w

    @pl.loop(0, ns_w // CH2)
    def _(i):
        base = base_t + i * CH2
        pltpu.sync_copy(sg0_hbm.at[pl.ds(base, CH2)], sidx[0])
        pltpu.sync_copy(sg1_hbm.at[pl.ds(base, CH2)], sidx[1])
        cpa = pltpu.async_copy(p_hbm.at[sidx[0]], rows[0], sem)
        cpb = pltpu.async_copy(q_hbm.at[sidx[1]], rows[1], la[0])
        cpa.wait()
        cpb.wait()

        @pl.loop(0, CH2)
        def _(r):
            for j in range(FC):
                rows[0][r, pl.ds(j * L, L)] = (
                    rows[0][r, pl.ds(j * L, L)]
                    + rows[1][r, pl.ds(j * L, L)])

        pltpu.sync_copy(rows[0], t_out.at[pl.ds(base, CH2)])

    plsc.subcore_barrier()

    nb_w = NBP // NW
    base_b = wid * nb_w
    n_a = nb_w // CH2

    def pref_a(i, b):
        base = base_b + i * CH2
        pltpu.async_copy(b1_hbm.at[pl.ds(base, CH2)], sidx[b], si[b])
        pltpu.async_copy(b0_hbm.at[pl.ds(base, CH2)], didx[b], di[b])
        pltpu.async_copy(bw_hbm.at[pl.ds(base, CH2)], wv[b], ws[b])

    def drain_a(b):
        pltpu.make_async_copy(b1_hbm.at[pl.ds(0, CH2)], sidx[b], si[b]).wait()
        pltpu.make_async_copy(b0_hbm.at[pl.ds(0, CH2)], didx[b], di[b]).wait()
        pltpu.make_async_copy(bw_hbm.at[pl.ds(0, CH2)], wv[b], ws[b]).wait()

    pref_a(0, 0)

    @pl.loop(0, n_a, step=2)
    def _(g):
        for b in range(2):
            i = g + b
            nb = 1 - b
            if b == 0:
                pref_a(i + 1, nb)
            else:
                @pl.when(g < n_a - 2)
                def _():
                    pref_a(i + 1, nb)
            drain_a(b)
            pltpu.async_copy(sn_hbm.at[sidx[b]], rows[b], sem).wait()
            _scale_rows(rows[b], wv[b], CH2)
            pltpu.sync_copy(rows[b], acc.at[didx[b]], add=True)

    plsc.subcore_barrier()
    pltpu.sync_copy(acc.at[pl.ds(row0, 632)], smsg_out.at[c, pl.ds(row0, 632)])

    zero_acc()
    plsc.subcore_barrier()

    base_e = wid * _ED_W

    def pref_e(i, b):
        base = base_e + i * CH2
        pltpu.async_copy(edges_hbm.at[pl.ds(base, CH2)], rows[b], la[b])
        pltpu.async_copy(g1_hbm.at[pl.ds(base, CH2)], didx[b], di[b])

    def drain_e(b):
        pltpu.make_async_copy(edges_hbm.at[pl.ds(0, CH2)], rows[b],
                              la[b]).wait()
        pltpu.make_async_copy(g1_hbm.at[pl.ds(0, CH2)], didx[b],
                              di[b]).wait()

    pref_e(0, 0)

    @pl.loop(0, _ED2_FULL, step=2)
    def _(g):
        for b in range(2):
            i = g + b
            nb = 1 - b
            if b == 0:
                pref_e(i + 1, nb)
            else:
                @pl.when(g < _ED2_FULL - 2)
                def _():
                    pref_e(i + 1, nb)
            drain_e(b)
            pltpu.sync_copy(rows[b], acc.at[didx[b]], add=True)

    tbase = base_e + _ED2_FULL * CH2
    pltpu.sync_copy(edges_hbm.at[pl.ds(tbase, _ED_TAIL)],
                    rows[0].at[pl.ds(0, _ED_TAIL)])
    pltpu.sync_copy(g1_hbm.at[pl.ds(tbase, _ED_TAIL)], didx_t)
    pltpu.sync_copy(rows[0].at[pl.ds(0, _ED_TAIL)], acc.at[didx_t], add=True)

    plsc.subcore_barrier()
    pltpu.sync_copy(acc.at[pl.ds(row0, 632)], emsg_out.at[c, pl.ds(row0, 632)])


@functools.cache
def _build_sc4():
    return functools.partial(
        pl.kernel,
        out_type=(jax.ShapeDtypeStruct((N_EDGES, D), _f32),
                  jax.ShapeDtypeStruct((N_EDGES, D), _f32)),
        mesh=_mesh(),
        scratch_types=[
            pltpu.VMEM((_ED_W,), _i32),
            pltpu.VMEM((_ED_W,), _i32),
            [pltpu.VMEM((CH, D), _f32)] * 3,
            [pltpu.VMEM((CH, D), _f32)] * 3,
            [pltpu.SemaphoreType.DMA] * 3,
            [pltpu.SemaphoreType.DMA] * 3,
            [pltpu.SemaphoreType.DMA] * 3,
            [pltpu.SemaphoreType.DMA] * 3,
        ],
    )(_sc4_body)


def _sc4_body(nodes_hbm, g0_hbm, g1_hbm, na_out, nb_out,
              idx0, idx1, rowsa, rowsb, ga, gb, wa, wb):
    c = lax.axis_index("c")
    s = lax.axis_index("s")
    wid = s * NC + c
    base_e = wid * _ED_W

    pltpu.sync_copy(g0_hbm.at[pl.ds(base_e, _ED_W)], idx0)
    pltpu.sync_copy(g1_hbm.at[pl.ds(base_e, _ED_W)], idx1)

    @pl.loop(0, _ED_FULL, step=3)
    def _(g):
        for b in range(3):
            i = g + b
            db = (b + 1) % 3
            base = base_e + i * CH
            cpa = pltpu.async_copy(
                nodes_hbm.at[idx0.at[pl.ds(i * CH, CH)]], rowsa[b], ga[b])
            cpb = pltpu.async_copy(
                nodes_hbm.at[idx1.at[pl.ds(i * CH, CH)]], rowsb[b], gb[b])

            def drain_prev():
                pltpu.make_async_copy(rowsa[db], na_out.at[pl.ds(0, CH)],
                                      wa[db]).wait()
                pltpu.make_async_copy(rowsb[db], nb_out.at[pl.ds(0, CH)],
                                      wb[db]).wait()

            if b < 2:
                @pl.when(g > 0)
                def _():
                    drain_prev()
            else:
                drain_prev()
            cpa.wait()
            cpb.wait()
            pltpu.async_copy(rowsa[b], na_out.at[pl.ds(base, CH)], wa[b])
            pltpu.async_copy(rowsb[b], nb_out.at[pl.ds(base, CH)], wb[b])

    for b in (1, 2):
        pltpu.make_async_copy(rowsa[b], na_out.at[pl.ds(0, CH)], wa[b]).wait()
        pltpu.make_async_copy(rowsb[b], nb_out.at[pl.ds(0, CH)], wb[b]).wait()

    tbase = base_e + _ED_FULL * CH
    pltpu.async_copy(nodes_hbm.at[idx0.at[pl.ds(_ED_FULL * CH, _ED_TAIL)]],
                     rowsa[0].at[pl.ds(0, _ED_TAIL)], ga[0]).wait()
    pltpu.async_copy(nodes_hbm.at[idx1.at[pl.ds(_ED_FULL * CH, _ED_TAIL)]],
                     rowsb[0].at[pl.ds(0, _ED_TAIL)], gb[0]).wait()
    pltpu.sync_copy(rowsa[0].at[pl.ds(0, _ED_TAIL)],
                    na_out.at[pl.ds(tbase, _ED_TAIL)])
    pltpu.sync_copy(rowsb[0].at[pl.ds(0, _ED_TAIL)],
                    nb_out.at[pl.ds(tbase, _ED_TAIL)])


def _dot(a, b):
    return jnp.dot(a, b, preferred_element_type=_f32)


def _sn_body(x, a0, a1, n0, n1, w1x, w1a, w1n, b1, w2, b2, pa, pb, bp,
             xo, po, qo):
    att = a0[0] + a1[0]
    nm = n0[0] + n1[0]
    h = jnp.maximum(
        _dot(x[...], w1x[...]) + _dot(att, w1a[...]) + _dot(nm, w1n[...])
        + b1[...], 0.0)
    xn = jnp.maximum(_dot(h, w2[...]) + b2[...], 0.0) + x[...]
    xo[...] = xn
    po[...] = _dot(xn, pa[...]) + bp[...]
    qo[...] = _dot(xn, pb[...])


def _nn_body(x, e0, e1, s0, s1, w1x, w1e, w1s, b1, w2, b2, xo):
    em = e0[0] + e1[0]
    sm = s0[0] + s1[0]
    h = jnp.maximum(
        _dot(x[...], w1x[...]) + _dot(em, w1e[...]) + _dot(sm, w1s[...])
        + b1[...], 0.0)
    xo[...] = jnp.maximum(_dot(h, w2[...]) + b2[...], 0.0) + x[...]


def _dotb(a, b):
    return jnp.dot(a, b, preferred_element_type=_f32)


def _se_body(t, e, c, w2, b2, out):
    h = jnp.maximum(t[...] + _dotb(e[...], c[...]), 0.0)
    out[...] = jnp.tanh(_dotb(h, w2[...]) + b2[...]) + e[...]


def _en_body(na, nb, e, w1a, w1b, w1c, b1, w2, b2, out):
    h = jnp.maximum(
        _dotb(na[...], w1a[...]) + _dotb(nb[...], w1b[...])
        + _dotb(e[...], w1c[...]) + b1[...], 0.0)
    out[...] = jnp.tanh(_dotb(h, w2[...]) + b2[...]) + e[...]


def _row_spec(rows):
    return pl.BlockSpec((rows, D), lambda i: (i, 0))


def _w_spec(shape):
    return pl.BlockSpec(shape, lambda i: tuple(0 for _ in shape))


def _part_spec(rows, core):
    return pl.BlockSpec((1, rows, D), lambda i, _c=core: (_c, i, 0))


def _tc_node_mlp(n, rows, x, msg1, msg2, w1, b1, w2, b2):
    grid = (n // rows,)
    ws = _w_spec((D, D))
    bs = _w_spec((1, D))
    ps0 = _part_spec(rows, 0)
    ps1 = _part_spec(rows, 1)
    return pl.pallas_call(
        _nn_body,
        grid=grid,
        in_specs=[_row_spec(rows), ps0, ps1, ps0, ps1, ws, ws, ws, bs, ws, bs],
        out_specs=_row_spec(rows),
        out_shape=jax.ShapeDtypeStruct((n, D), _f32),
        compiler_params=pltpu.CompilerParams(
            dimension_semantics=("arbitrary",)),
    )(x, msg1, msg1, msg2, msg2, w1[:D], w1[D:2 * D], w1[2 * D:],
      b1.reshape(1, D), w2, b2.reshape(1, D))


def _tc_sn_mlp(x, amsg, nmsg, w1, b1, w2, b2, pa, pb, bp):
    outs = [jax.ShapeDtypeStruct((N_SUPER, D), _f32)] * 3
    xs = pl.BlockSpec((N_SUPER, D), lambda i: (0, 0))
    ws = pl.BlockSpec((D, D), lambda i: (0, 0))
    bs = pl.BlockSpec((1, D), lambda i: (0, 0))
    ps0 = pl.BlockSpec((1, N_SUPER, D), lambda i: (0, 0, 0))
    ps1 = pl.BlockSpec((1, N_SUPER, D), lambda i: (1, 0, 0))
    return pl.pallas_call(
        _sn_body,
        grid=(1,),
        in_specs=[xs, ps0, ps1, ps0, ps1, ws, ws, ws, bs, ws, bs, ws, ws, bs],
        out_specs=[xs] * 3,
        out_shape=outs,
    )(x, amsg, amsg, nmsg, nmsg, w1[:D], w1[D:2 * D], w1[2 * D:],
      b1.reshape(1, D), w2, b2.reshape(1, D), pa, pb, bp.reshape(1, D))


def _tc_se_mlp(t, e, c, w2, b2):
    rows = 2000
    grid = (N_SED // rows,)
    ws = _w_spec((D, D))
    bs = _w_spec((1, D))
    return pl.pallas_call(
        _se_body,
        grid=grid,
        in_specs=[_row_spec(rows), _row_spec(rows), ws, ws, bs],
        out_specs=_row_spec(rows),
        out_shape=jax.ShapeDtypeStruct((N_SED, D), _f32),
        compiler_params=pltpu.CompilerParams(
            dimension_semantics=("arbitrary",)),
    )(t, e, c, w2, b2.reshape(1, D))


def _tc_en_mlp(na, nb, e, w1, b1, w2, b2):
    rows = 4000
    grid = (N_EDGES // rows,)
    ws = _w_spec((D, D))
    bs = _w_spec((1, D))
    return pl.pallas_call(
        _en_body,
        grid=grid,
        in_specs=[_row_spec(rows)] * 3 + [ws, ws, ws, bs, ws, bs],
        out_specs=_row_spec(rows),
        out_shape=jax.ShapeDtypeStruct((N_EDGES, D), _f32),
        compiler_params=pltpu.CompilerParams(
            dimension_semantics=("arbitrary",)),
    )(na, nb, e, w1[:D], w1[D:2 * D], w1[2 * D:], b1.reshape(1, D),
      w2, b2.reshape(1, D))


def kernel(nodes, edges, supernodes, superedges, graph, bipartite_graph,
           bipartite_edge_weights, super_graph, super_edge_weights,
           en_W1, en_b1, en_W2, en_b2, nn_W1, nn_b1, nn_W2, nn_b2,
           sn_W1, sn_b1, sn_W2, sn_b2, se_W1, se_b1, se_W2, se_b2):
    g0 = graph[0]
    g1 = graph[1]
    b0 = jnp.pad(bipartite_graph[0], (0, NBP - N_BIP))
    b1i = jnp.pad(bipartite_graph[1], (0, NBP - N_BIP))
    bw = jnp.broadcast_to(
        jnp.pad(bipartite_edge_weights, ((0, NBP - N_BIP), (0, 0))), (NBP, L))
    sedp = jnp.pad(superedges, ((0, NSP - N_SED), (0, 0)))
    sw = jnp.broadcast_to(
        jnp.pad(super_edge_weights, ((0, NSP - N_SED), (0, 0))), (NSP, L))
    sg0 = jnp.pad(super_graph[0], (0, NSP - N_SED))
    sg1 = jnp.pad(super_graph[1], (0, NSP - N_SED))

    nmsg, amsg = _build_sc1()(nodes, b0, b1i, bw, sedp, sw, sg1)

    sn_new, p_se, q_se = _tc_sn_mlp(
        supernodes, amsg, nmsg, sn_W1, sn_b1, sn_W2, sn_b2,
        se_W1[:D], se_W1[D:2 * D], se_b1)

    smsg, emsg, t_se = _build_sc2()(
        sn_new, b0, b1i, bw, edges, g1, p_se, q_se, sg0, sg1)

    nodes_new = _tc_node_mlp(
        N_NODES, 2000, nodes, emsg, smsg, nn_W1, nn_b1, nn_W2, nn_b2)

    sed_new = _tc_se_mlp(t_se, superedges, se_W1[2 * D:], se_W2, se_b2)
    na, nb = _build_sc4()(nodes_new, g0, g1)
    edges_new = _tc_en_mlp(na, nb, edges, en_W1, en_b1, en_W2, en_b2)

    return (nodes_new, edges_new, sn_new, sed_new)


# speedup vs baseline: 1.1398x; 1.0050x over previous
import functools

import jax
import jax.numpy as jnp
from jax import lax
from jax.experimental import pallas as pl
from jax.experimental.pallas import tpu as pltpu
from jax.experimental.pallas import tpu_sc as plsc

D = 128
L = 16
FC = D // L
NC = 2
NSUB = 16
NW = NC * NSUB

N_NODES = 10000
N_EDGES = 320000
N_SUPER = 1000
N_BIP = 40000
N_SED = 16000
NBP = 40960
NSP = 16384
CH = 128

@functools.cache
def _mesh():
    return plsc.VectorSubcoreMesh(
        core_axis_name="c", subcore_axis_name="s",
        num_cores=NC, num_subcores=NSUB)


_f32 = jnp.float32
_i32 = jnp.int32


def _zero_vmem(buf, nrows):
    z = jnp.zeros((L,), _f32)

    @pl.loop(0, nrows)
    def _(r):
        for j in range(FC):
            buf[r, pl.ds(j * L, L)] = z


def _scale_rows(rows, wv, nrows):
    @pl.loop(0, nrows)
    def _(r):
        wr = wv[r, :]
        for j in range(FC):
            rows[r, pl.ds(j * L, L)] = rows[r, pl.ds(j * L, L)] * wr


@functools.cache
def _build_sc1():
    return functools.partial(
        pl.kernel,
        out_type=(jax.ShapeDtypeStruct((NC, 1024, D), _f32),
                  jax.ShapeDtypeStruct((NC, 1024, D), _f32)),
        mesh=_mesh(),
        scratch_types=[
            [pltpu.VMEM((CH,), _i32)] * 2,
            [pltpu.VMEM((CH,), _i32)] * 2,
            [pltpu.VMEM((CH, L), _f32)] * 2,
            [pltpu.VMEM((CH, D), _f32)] * 2,
            pltpu.VMEM((64, D), _f32),
            pltpu.VMEM_SHARED((1024, D), _f32),
            pltpu.VMEM_SHARED((1024, D), _f32),
            pltpu.SemaphoreType.DMA,
            [pltpu.SemaphoreType.DMA] * 2,
            [pltpu.SemaphoreType.DMA] * 2,
            [pltpu.SemaphoreType.DMA] * 2,
            [pltpu.SemaphoreType.DMA] * 2,
        ],
    )(_sc1_body)


def _sc1_body(nodes_hbm, b0_hbm, b1_hbm, bw_hbm, sed_hbm, sw_hbm, sg1_hbm,
              nmsg_out, amsg_out, sidx, didx, wv, rows, zbuf, acc_n, acc_a,
              sem, si, di, ws, la):
    c = lax.axis_index("c")
    s = lax.axis_index("s")
    wid = s * NC + c

    _zero_vmem(zbuf, 64)
    pltpu.sync_copy(zbuf, acc_n.at[pl.ds(s * 64, 64)])
    pltpu.sync_copy(zbuf, acc_a.at[pl.ds(s * 64, 64)])
    plsc.subcore_barrier()

    nb_w = NBP // NW
    base_b = wid * nb_w
    n_a = nb_w // CH

    def pref_a(i, b):
        base = base_b + i * CH
        pltpu.async_copy(b0_hbm.at[pl.ds(base, CH)], sidx[b], si[b])
        pltpu.async_copy(b1_hbm.at[pl.ds(base, CH)], didx[b], di[b])
        pltpu.async_copy(bw_hbm.at[pl.ds(base, CH)], wv[b], ws[b])

    def drain_a(b):
        pltpu.make_async_copy(b0_hbm.at[pl.ds(0, CH)], sidx[b], si[b]).wait()
        pltpu.make_async_copy(b1_hbm.at[pl.ds(0, CH)], didx[b], di[b]).wait()
        pltpu.make_async_copy(bw_hbm.at[pl.ds(0, CH)], wv[b], ws[b]).wait()

    pref_a(0, 0)

    @pl.loop(0, n_a, step=2)
    def _(g):
        for b in range(2):
            i = g + b
            nb = 1 - b
            if b == 0:
                pref_a(i + 1, nb)
            else:
                @pl.when(g < n_a - 2)
                def _():
                    pref_a(i + 1, nb)
            drain_a(b)
            pltpu.async_copy(nodes_hbm.at[sidx[b]], rows[b], sem).wait()
            _scale_rows(rows[b], wv[b], CH)
            pltpu.sync_copy(rows[b], acc_n.at[didx[b]], add=True)

    ns_w = NSP // NW
    base_s = wid * ns_w
    n_b = ns_w // CH

    def pref_b(i, b):
        base = base_s + i * CH
        pltpu.async_copy(sed_hbm.at[pl.ds(base, CH)], rows[b], la[b])
        pltpu.async_copy(sg1_hbm.at[pl.ds(base, CH)], didx[b], di[b])
        pltpu.async_copy(sw_hbm.at[pl.ds(base, CH)], wv[b], ws[b])

    def drain_b(b):
        pltpu.make_async_copy(sed_hbm.at[pl.ds(0, CH)], rows[b], la[b]).wait()
        pltpu.make_async_copy(sg1_hbm.at[pl.ds(0, CH)], didx[b], di[b]).wait()
        pltpu.make_async_copy(sw_hbm.at[pl.ds(0, CH)], wv[b], ws[b]).wait()

    pref_b(0, 0)

    @pl.loop(0, n_b, step=2)
    def _(g):
        for b in range(2):
            i = g + b
            nb = 1 - b
            if b == 0:
                pref_b(i + 1, nb)
            else:
                @pl.when(g < n_b - 2)
                def _():
                    pref_b(i + 1, nb)
            drain_b(b)
            _scale_rows(rows[b], wv[b], CH)
            pltpu.sync_copy(rows[b], acc_a.at[didx[b]], add=True)

    plsc.subcore_barrier()
    pltpu.sync_copy(acc_n.at[pl.ds(s * 64, 64)], nmsg_out.at[c, pl.ds(s * 64, 64)])
    pltpu.sync_copy(acc_a.at[pl.ds(s * 64, 64)], amsg_out.at[c, pl.ds(s * 64, 64)])


N_ACC = 10112
CH2 = 64
_ED_W = N_EDGES // NW
_ED_FULL = _ED_W // CH
_ED_TAIL = _ED_W - _ED_FULL * CH
_ED2_FULL = _ED_W // CH2


@functools.cache
def _build_sc2():
    return functools.partial(
        pl.kernel,
        out_type=(jax.ShapeDtypeStruct((NC, N_ACC, D), _f32),
                  jax.ShapeDtypeStruct((NC, N_ACC, D), _f32),
                  jax.ShapeDtypeStruct((NSP, D), _f32)),
        mesh=_mesh(),
        scratch_types=[
            [pltpu.VMEM((CH2,), _i32)] * 2,
            [pltpu.VMEM((CH2,), _i32)] * 2,
            pltpu.VMEM((_ED_TAIL,), _i32),
            [pltpu.VMEM((CH2, L), _f32)] * 2,
            [pltpu.VMEM((CH2, D), _f32)] * 2,
            pltpu.VMEM_SHARED((N_ACC, D), _f32),
            pltpu.SemaphoreType.DMA,
            [pltpu.SemaphoreType.DMA] * 2,
            [pltpu.SemaphoreType.DMA] * 2,
            [pltpu.SemaphoreType.DMA] * 2,
            [pltpu.SemaphoreType.DMA] * 2,
        ],
    )(_sc2_body)


def _sc2_body(sn_hbm, b0_hbm, b1_hbm, bw_hbm, edges_hbm, g1_hbm,
              p_hbm, q_hbm, sg0_hbm, sg1_hbm,
              smsg_out, emsg_out, t_out, sidx, didx, didx_t, wv, rows, acc,
              sem, si, di, ws, la):
    c = lax.axis_index("c")
    s = lax.axis_index("s")
    wid = s * NC + c
    row0 = s * 632

    def zero_acc():
        _zero_vmem(rows[0], CH2)
        for t in range(9):
            pltpu.sync_copy(rows[0], acc.at[pl.ds(row0 + t * 64, 64)])
        pltpu.sync_copy(rows[0].at[pl.ds(0, 56)],
                        acc.at[pl.ds(row0 + 576, 56)])

    zero_acc()

    ns_w = NSP // NW
    base_t = wid * ns_w

    @pl.loop(0, ns_w // CH2)
    def _(i):
        base = base_t + i * CH2
        pltpu.sync_copy(sg0_hbm.at[pl.ds(base, CH2)], sidx[0])
        pltpu.sync_copy(sg1_hbm.at[pl.ds(base, CH2)], sidx[1])
        cpa = pltpu.async_copy(p_hbm.at[sidx[0]], rows[0], sem)
        cpb = pltpu.async_copy(q_hbm.at[sidx[1]], rows[1], la[0])
        cpa.wait()
        cpb.wait()

        @pl.loop(0, CH2)
        def _(r):
            for j in range(FC):
                rows[0][r, pl.ds(j * L, L)] = (
                    rows[0][r, pl.ds(j * L, L)]
                    + rows[1][r, pl.ds(j * L, L)])

        pltpu.sync_copy(rows[0], t_out.at[pl.ds(base, CH2)])

    plsc.subcore_barrier()

    nb_w = NBP // NW
    base_b = wid * nb_w
    n_a = nb_w // CH2

    def pref_a(i, b):
        base = base_b + i * CH2
        pltpu.async_copy(b1_hbm.at[pl.ds(base, CH2)], sidx[b], si[b])
        pltpu.async_copy(b0_hbm.at[pl.ds(base, CH2)], didx[b], di[b])
        pltpu.async_copy(bw_hbm.at[pl.ds(base, CH2)], wv[b], ws[b])

    def drain_a(b):
        pltpu.make_async_copy(b1_hbm.at[pl.ds(0, CH2)], sidx[b], si[b]).wait()
        pltpu.make_async_copy(b0_hbm.at[pl.ds(0, CH2)], didx[b], di[b]).wait()
        pltpu.make_async_copy(bw_hbm.at[pl.ds(0, CH2)], wv[b], ws[b]).wait()

    pref_a(0, 0)

    @pl.loop(0, n_a, step=2)
    def _(g):
        for b in range(2):
            i = g + b
            nb = 1 - b
            if b == 0:
                pref_a(i + 1, nb)
            else:
                @pl.when(g < n_a - 2)
                def _():
                    pref_a(i + 1, nb)
            drain_a(b)
            pltpu.async_copy(sn_hbm.at[sidx[b]], rows[b], sem).wait()
            _scale_rows(rows[b], wv[b], CH2)
            pltpu.sync_copy(rows[b], acc.at[didx[b]], add=True)

    plsc.subcore_barrier()
    pltpu.sync_copy(acc.at[pl.ds(row0, 632)], smsg_out.at[c, pl.ds(row0, 632)])

    zero_acc()
    plsc.subcore_barrier()

    base_e = wid * _ED_W

    def pref_e(i, b):
        base = base_e + i * CH2
        pltpu.async_copy(edges_hbm.at[pl.ds(base, CH2)], rows[b], la[b])
        pltpu.async_copy(g1_hbm.at[pl.ds(base, CH2)], didx[b], di[b])

    def drain_e(b):
        pltpu.make_async_copy(edges_hbm.at[pl.ds(0, CH2)], rows[b],
                              la[b]).wait()
        pltpu.make_async_copy(g1_hbm.at[pl.ds(0, CH2)], didx[b],
                              di[b]).wait()

    pref_e(0, 0)

    @pl.loop(0, _ED2_FULL, step=2)
    def _(g):
        for b in range(2):
            i = g + b
            nb = 1 - b
            if b == 0:
                pref_e(i + 1, nb)
            else:
                @pl.when(g < _ED2_FULL - 2)
                def _():
                    pref_e(i + 1, nb)
            drain_e(b)
            pltpu.sync_copy(rows[b], acc.at[didx[b]], add=True)

    tbase = base_e + _ED2_FULL * CH2
    pltpu.sync_copy(edges_hbm.at[pl.ds(tbase, _ED_TAIL)],
                    rows[0].at[pl.ds(0, _ED_TAIL)])
    pltpu.sync_copy(g1_hbm.at[pl.ds(tbase, _ED_TAIL)], didx_t)
    pltpu.sync_copy(rows[0].at[pl.ds(0, _ED_TAIL)], acc.at[didx_t], add=True)

    plsc.subcore_barrier()
    pltpu.sync_copy(acc.at[pl.ds(row0, 632)], emsg_out.at[c, pl.ds(row0, 632)])


@functools.cache
def _build_sc4():
    return functools.partial(
        pl.kernel,
        out_type=(jax.ShapeDtypeStruct((N_EDGES, D), _f32),
                  jax.ShapeDtypeStruct((N_EDGES, D), _f32)),
        mesh=_mesh(),
        scratch_types=[
            pltpu.VMEM((_ED_W,), _i32),
            pltpu.VMEM((_ED_W,), _i32),
            [pltpu.VMEM((CH, D), _f32)] * 3,
            [pltpu.VMEM((CH, D), _f32)] * 3,
            [pltpu.SemaphoreType.DMA] * 3,
            [pltpu.SemaphoreType.DMA] * 3,
            [pltpu.SemaphoreType.DMA] * 3,
            [pltpu.SemaphoreType.DMA] * 3,
        ],
    )(_sc4_body)


def _sc4_body(nodes_hbm, g0_hbm, g1_hbm, na_out, nb_out,
              idx0, idx1, rowsa, rowsb, ga, gb, wa, wb):
    c = lax.axis_index("c")
    s = lax.axis_index("s")
    wid = s * NC + c
    base_e = wid * _ED_W

    pltpu.sync_copy(g0_hbm.at[pl.ds(base_e, _ED_W)], idx0)
    pltpu.sync_copy(g1_hbm.at[pl.ds(base_e, _ED_W)], idx1)

    @pl.loop(0, _ED_FULL, step=3)
    def _(g):
        for b in range(3):
            i = g + b
            db = (b + 1) % 3
            base = base_e + i * CH
            cpa = pltpu.async_copy(
                nodes_hbm.at[idx0.at[pl.ds(i * CH, CH)]], rowsa[b], ga[b])
            cpb = pltpu.async_copy(
                nodes_hbm.at[idx1.at[pl.ds(i * CH, CH)]], rowsb[b], gb[b])

            def drain_prev():
                pltpu.make_async_copy(rowsa[db], na_out.at[pl.ds(0, CH)],
                                      wa[db]).wait()
                pltpu.make_async_copy(rowsb[db], nb_out.at[pl.ds(0, CH)],
                                      wb[db]).wait()

            if b < 2:
                @pl.when(g > 0)
                def _():
                    drain_prev()
            else:
                drain_prev()
            cpa.wait()
            cpb.wait()
            pltpu.async_copy(rowsa[b], na_out.at[pl.ds(base, CH)], wa[b])
            pltpu.async_copy(rowsb[b], nb_out.at[pl.ds(base, CH)], wb[b])

    for b in (1, 2):
        pltpu.make_async_copy(rowsa[b], na_out.at[pl.ds(0, CH)], wa[b]).wait()
        pltpu.make_async_copy(rowsb[b], nb_out.at[pl.ds(0, CH)], wb[b]).wait()

    tbase = base_e + _ED_FULL * CH
    pltpu.async_copy(nodes_hbm.at[idx0.at[pl.ds(_ED_FULL * CH, _ED_TAIL)]],
                     rowsa[0].at[pl.ds(0, _ED_TAIL)], ga[0]).wait()
    pltpu.async_copy(nodes_hbm.at[idx1.at[pl.ds(_ED_FULL * CH, _ED_TAIL)]],
                     rowsb[0].at[pl.ds(0, _ED_TAIL)], gb[0]).wait()
    pltpu.sync_copy(rowsa[0].at[pl.ds(0, _ED_TAIL)],
                    na_out.at[pl.ds(tbase, _ED_TAIL)])
    pltpu.sync_copy(rowsb[0].at[pl.ds(0, _ED_TAIL)],
                    nb_out.at[pl.ds(tbase, _ED_TAIL)])


def _dot(a, b):
    return jnp.dot(a, b, preferred_element_type=_f32)


def _sn_body(x, a0, a1, n0, n1, w1x, w1a, w1n, b1, w2, b2, pa, pb, bp,
             xo, po, qo):
    att = a0[0] + a1[0]
    nm = n0[0] + n1[0]
    h = jnp.maximum(
        _dot(x[...], w1x[...]) + _dot(att, w1a[...]) + _dot(nm, w1n[...])
        + b1[...], 0.0)
    xn = jnp.maximum(_dot(h, w2[...]) + b2[...], 0.0) + x[...]
    xo[...] = xn
    po[...] = _dot(xn, pa[...]) + bp[...]
    qo[...] = _dot(xn, pb[...])


def _nn_body(x, e0, e1, s0, s1, w1x, w1e, w1s, b1, w2, b2, xo):
    em = e0[0] + e1[0]
    sm = s0[0] + s1[0]
    h = jnp.maximum(
        _dot(x[...], w1x[...]) + _dot(em, w1e[...]) + _dot(sm, w1s[...])
        + b1[...], 0.0)
    xo[...] = jnp.maximum(_dot(h, w2[...]) + b2[...], 0.0) + x[...]


def _dotb(a, b):
    return jnp.dot(a, b, preferred_element_type=_f32)


def _se_body(t, e, c, w2, b2, out):
    h = jnp.maximum(t[...] + _dotb(e[...], c[...]), 0.0)
    out[...] = jnp.tanh(_dotb(h, w2[...]) + b2[...]) + e[...]


def _en_body(na, nb, e, w1a, w1b, w1c, b1, w2, b2, out):
    h = jnp.maximum(
        _dotb(na[...], w1a[...]) + _dotb(nb[...], w1b[...])
        + _dotb(e[...], w1c[...]) + b1[...], 0.0)
    out[...] = jnp.tanh(_dotb(h, w2[...]) + b2[...]) + e[...]


def _row_spec(rows):
    return pl.BlockSpec((rows, D), lambda i: (i, 0))


def _w_spec(shape):
    return pl.BlockSpec(shape, lambda i: tuple(0 for _ in shape))


def _part_spec(rows, core):
    return pl.BlockSpec((1, rows, D), lambda i, _c=core: (_c, i, 0))


def _tc_node_mlp(n, rows, x, msg1, msg2, w1, b1, w2, b2):
    grid = (n // rows,)
    ws = _w_spec((D, D))
    bs = _w_spec((1, D))
    ps0 = _part_spec(rows, 0)
    ps1 = _part_spec(rows, 1)
    return pl.pallas_call(
        _nn_body,
        grid=grid,
        in_specs=[_row_spec(rows), ps0, ps1, ps0, ps1, ws, ws, ws, bs, ws, bs],
        out_specs=_row_spec(rows),
        out_shape=jax.ShapeDtypeStruct((n, D), _f32),
        compiler_params=pltpu.CompilerParams(
            dimension_semantics=("arbitrary",)),
    )(x, msg1, msg1, msg2, msg2, w1[:D], w1[D:2 * D], w1[2 * D:],
      b1.reshape(1, D), w2, b2.reshape(1, D))


def _tc_sn_mlp(x, amsg, nmsg, w1, b1, w2, b2, pa, pb, bp):
    outs = [jax.ShapeDtypeStruct((N_SUPER, D), _f32)] * 3
    xs = pl.BlockSpec((N_SUPER, D), lambda i: (0, 0))
    ws = pl.BlockSpec((D, D), lambda i: (0, 0))
    bs = pl.BlockSpec((1, D), lambda i: (0, 0))
    ps0 = pl.BlockSpec((1, N_SUPER, D), lambda i: (0, 0, 0))
    ps1 = pl.BlockSpec((1, N_SUPER, D), lambda i: (1, 0, 0))
    return pl.pallas_call(
        _sn_body,
        grid=(1,),
        in_specs=[xs, ps0, ps1, ps0, ps1, ws, ws, ws, bs, ws, bs, ws, ws, bs],
        out_specs=[xs] * 3,
        out_shape=outs,
    )(x, amsg, amsg, nmsg, nmsg, w1[:D], w1[D:2 * D], w1[2 * D:],
      b1.reshape(1, D), w2, b2.reshape(1, D), pa, pb, bp.reshape(1, D))


def _tc_se_mlp(t, e, c, w2, b2):
    rows = 4000
    grid = (N_SED // rows,)
    ws = _w_spec((D, D))
    bs = _w_spec((1, D))
    return pl.pallas_call(
        _se_body,
        grid=grid,
        in_specs=[_row_spec(rows), _row_spec(rows), ws, ws, bs],
        out_specs=_row_spec(rows),
        out_shape=jax.ShapeDtypeStruct((N_SED, D), _f32),
        compiler_params=pltpu.CompilerParams(
            dimension_semantics=("arbitrary",)),
    )(t, e, c, w2, b2.reshape(1, D))


def _tc_en_mlp(na, nb, e, w1, b1, w2, b2):
    rows = 8000
    grid = (N_EDGES // rows,)
    ws = _w_spec((D, D))
    bs = _w_spec((1, D))
    return pl.pallas_call(
        _en_body,
        grid=grid,
        in_specs=[_row_spec(rows)] * 3 + [ws, ws, ws, bs, ws, bs],
        out_specs=_row_spec(rows),
        out_shape=jax.ShapeDtypeStruct((N_EDGES, D), _f32),
        compiler_params=pltpu.CompilerParams(
            dimension_semantics=("arbitrary",)),
    )(na, nb, e, w1[:D], w1[D:2 * D], w1[2 * D:], b1.reshape(1, D),
      w2, b2.reshape(1, D))


def kernel(nodes, edges, supernodes, superedges, graph, bipartite_graph,
           bipartite_edge_weights, super_graph, super_edge_weights,
           en_W1, en_b1, en_W2, en_b2, nn_W1, nn_b1, nn_W2, nn_b2,
           sn_W1, sn_b1, sn_W2, sn_b2, se_W1, se_b1, se_W2, se_b2):
    g0 = graph[0]
    g1 = graph[1]
    b0 = jnp.pad(bipartite_graph[0], (0, NBP - N_BIP))
    b1i = jnp.pad(bipartite_graph[1], (0, NBP - N_BIP))
    bw = jnp.broadcast_to(
        jnp.pad(bipartite_edge_weights, ((0, NBP - N_BIP), (0, 0))), (NBP, L))
    sedp = jnp.pad(superedges, ((0, NSP - N_SED), (0, 0)))
    sw = jnp.broadcast_to(
        jnp.pad(super_edge_weights, ((0, NSP - N_SED), (0, 0))), (NSP, L))
    sg0 = jnp.pad(super_graph[0], (0, NSP - N_SED))
    sg1 = jnp.pad(super_graph[1], (0, NSP - N_SED))

    nmsg, amsg = _build_sc1()(nodes, b0, b1i, bw, sedp, sw, sg1)

    sn_new, p_se, q_se = _tc_sn_mlp(
        supernodes, amsg, nmsg, sn_W1, sn_b1, sn_W2, sn_b2,
        se_W1[:D], se_W1[D:2 * D], se_b1)

    smsg, emsg, t_se = _build_sc2()(
        sn_new, b0, b1i, bw, edges, g1, p_se, q_se, sg0, sg1)

    nodes_new = _tc_node_mlp(
        N_NODES, 2000, nodes, emsg, smsg, nn_W1, nn_b1, nn_W2, nn_b2)

    sed_new = _tc_se_mlp(t_se, superedges, se_W1[2 * D:], se_W2, se_b2)
    na, nb = _build_sc4()(nodes_new, g0, g1)
    edges_new = _tc_en_mlp(na, nb, edges, en_W1, en_b1, en_W2, en_b2)

    return (nodes_new, edges_new, sn_new, sed_new)
